# Initial kernel scaffold; baseline (speedup 1.0000x reference)
import jax
import jax.numpy as jnp
from jax.experimental import pallas as pl


def kernel(x, adj, W1, b1, W2, b2):
    raise NotImplementedError("write your pallas kernel here")


# speedup vs baseline: 3.9119x; 3.9119x over previous
import functools

import jax
import jax.numpy as jnp
from jax import lax
from jax.experimental import pallas as pl
from jax.experimental.pallas import tpu as pltpu
from jax.experimental.pallas import tpu_sc as plsc

N = 10000
NPAD = 10240
F = 128
NCLASS = 64
THR = 0.1
NC = 2
NS = 16
NW = NC * NS
C = 128
NCH = 79
EPT = NCH * C
EPAD = NW * EPT
SL = NPAD // NS
f32 = jnp.float32
i32 = jnp.int32

_mesh = plsc.VectorSubcoreMesh(core_axis_name="c", subcore_axis_name="s")


def _make_sim_pass(has_prev):
    out_type = (jax.ShapeDtypeStruct((NW, NCH, C), f32),
                jax.ShapeDtypeStruct((NW, NPAD), f32))
    scratch = [
        pltpu.VMEM((NCH, C), i32),
        pltpu.VMEM((NCH, C), i32),
        pltpu.VMEM((C, F), f32),
        pltpu.VMEM((C, F), f32),
        pltpu.VMEM((C,), f32),
        pltpu.VMEM((NPAD,), f32),
        pltpu.SemaphoreType.DMA,
        pltpu.SemaphoreType.DMA,
    ]
    if has_prev:
        scratch.insert(2, pltpu.VMEM((NCH, C), f32))

    def body(*refs):
        if has_prev:
            (feat_hbm, rowp_hbm, colp_hbm, prev_hbm, val_hbm, deg_hbm,
             rowv, colv, prevv, rbuf, cbuf, valb, degl, sem1, sem2) = refs
        else:
            (feat_hbm, rowp_hbm, colp_hbm, val_hbm, deg_hbm,
             rowv, colv, rbuf, cbuf, valb, degl, sem1, sem2) = refs
        cid = lax.axis_index("c")
        sid = lax.axis_index("s")
        wid = sid * NC + cid
        pltpu.sync_copy(rowp_hbm.at[wid], rowv)
        pltpu.sync_copy(colp_hbm.at[wid], colv)
        if has_prev:
            pltpu.sync_copy(prev_hbm.at[wid], prevv)

        zero16 = jnp.zeros((16,), f32)

        def zbody(i, carry):
            degl[pl.ds(i * 16, 16)] = zero16
            return carry
        lax.fori_loop(0, NPAD // 16, zbody, 0)

        iota = lax.iota(i32, 16)

        def chunk(j, carry):
            cp1 = pltpu.async_copy(feat_hbm.at[rowv.at[j]], rbuf, sem1)
            cp2 = pltpu.async_copy(feat_hbm.at[colv.at[j]], cbuf, sem2)
            cp1.wait()
            cp2.wait()
            for g in range(C // 16):
                rows16 = iota + (g * 16)

                def dot(k, acc):
                    kk = jnp.full((16,), k, i32)
                    a = plsc.load_gather(rbuf, [rows16, kk])
                    b = plsc.load_gather(cbuf, [rows16, kk])
                    return acc + a * b
                sims = lax.fori_loop(0, F, dot, jnp.zeros((16,), f32),
                                     unroll=4)
                rv = rowv[j, pl.ds(g * 16, 16)]
                cv = colv[j, pl.ds(g * 16, 16)]
                m = (sims >= THR) & (rv != cv)
                if has_prev:
                    m = m & (prevv[j, pl.ds(g * 16, 16)] > 0.0)
                val = jnp.where(m, sims, 0.0)
                valb[pl.ds(g * 16, 16)] = val
                plsc.addupdate_scatter(degl, [rv], val)
            pltpu.sync_copy(valb, val_hbm.at[wid, j])
            return carry
        lax.fori_loop(0, NCH, chunk, 0)
        pltpu.sync_copy(degl, deg_hbm.at[wid])

    return pl.kernel(body, out_type=out_type, mesh=_mesh,
                     compiler_params=pltpu.CompilerParams(
                         needs_layout_passes=False),
                     scratch_types=scratch)


def _make_agg_pass(D, use_q):
    out_type = jax.ShapeDtypeStruct((NC, NPAD, D), f32)
    scratch = [
        pltpu.VMEM((C,), i32),
        pltpu.VMEM((C,), i32),
        pltpu.VMEM((C,), f32),
        pltpu.VMEM((C, D), f32),
        pltpu.VMEM((C,), f32),
        pltpu.VMEM((16, D), f32),
        pltpu.VMEM_SHARED((NPAD, D), f32),
        pltpu.SemaphoreType.DMA,
        pltpu.SemaphoreType.DMA,
    ]
    if use_q:
        scratch.insert(3, pltpu.VMEM((NPAD,), f32))

    def body(*refs):
        if use_q:
            (feat_hbm, rowp_hbm, colp_hbm, val_hbm, q_hbm, acc_hbm,
             rowc, colc, valc, qv, gbuf, wbuf, zbuf, acc_sh,
             sem1, sem2) = refs
        else:
            (feat_hbm, rowp_hbm, colp_hbm, val_hbm, acc_hbm,
             rowc, colc, valc, gbuf, wbuf, zbuf, acc_sh, sem1, sem2) = refs
        cid = lax.axis_index("c")
        sid = lax.axis_index("s")
        wid = sid * NC + cid
        if use_q:
            pltpu.sync_copy(q_hbm, qv)

        zero16 = jnp.zeros((16,), f32)

        def zrow(r, carry):
            for k in range(D // 16):
                zbuf[r, pl.ds(k * 16, 16)] = zero16
            return carry
        lax.fori_loop(0, 16, zrow, 0)

        def zacc(t, carry):
            pltpu.sync_copy(zbuf, acc_sh.at[pl.ds(sid * SL + t * 16, 16)])
            return carry
        lax.fori_loop(0, SL // 16, zacc, 0)
        plsc.subcore_barrier()

        def chunk(j, carry):
            pltpu.sync_copy(rowp_hbm.at[wid, j], rowc)
            pltpu.sync_copy(colp_hbm.at[wid, j], colc)
            pltpu.sync_copy(val_hbm.at[wid, j], valc)
            pltpu.async_copy(feat_hbm.at[colc], gbuf, sem1).wait()
            for g in range(C // 16):
                val = valc[pl.ds(g * 16, 16)]
                if use_q:
                    cv = colc[pl.ds(g * 16, 16)]
                    val = val * plsc.load_gather(qv, [cv])
                wbuf[pl.ds(g * 16, 16)] = val

            def scale(e, c2):
                wv = plsc.load_gather(wbuf, [jnp.full((16,), e, i32)])
                for k in range(D // 16):
                    gbuf[e, pl.ds(k * 16, 16)] = (
                        gbuf[e, pl.ds(k * 16, 16)] * wv)
                return c2
            lax.fori_loop(0, C, scale, 0)
            pltpu.sync_copy(gbuf, acc_sh.at[rowc], add=True)
            return carry
        lax.fori_loop(0, NCH, chunk, 0)
        plsc.subcore_barrier()

        def dump(t, carry):
            pltpu.sync_copy(acc_sh.at[pl.ds(sid * SL + t * 16, 16)], zbuf)
            pltpu.sync_copy(zbuf, acc_hbm.at[cid, pl.ds(sid * SL + t * 16, 16)])
            return carry
        lax.fori_loop(0, SL // 16, dump, 0)

    return pl.kernel(body, out_type=out_type, mesh=_mesh,
                     compiler_params=pltpu.CompilerParams(
                         needs_layout_passes=False,
                         use_tc_tiling_on_sc=(D % 128 == 0)),
                     scratch_types=scratch)


_sim_pass1 = _make_sim_pass(False)
_sim_pass2 = _make_sim_pass(True)
_agg_pass1 = _make_agg_pass(F, True)
_agg_pass2 = _make_agg_pass(NCLASS, False)


def _tc1_body(x_ref, xn_ref, nrc_ref):
    x = x_ref[...]
    nr = jnp.sqrt(jnp.sum(x * x, axis=1, keepdims=True))
    nrc = jnp.maximum(nr, 1e-8)
    xn_ref[...] = x / nrc
    nrc_ref[...] = jnp.concatenate(
        [nrc, jnp.ones((NPAD - N, 1), f32)], axis=0)


def _tc1(x):
    return pl.pallas_call(
        _tc1_body,
        out_shape=(jax.ShapeDtypeStruct((N, F), f32),
                   jax.ShapeDtypeStruct((NPAD, 1), f32)),
    )(x)


def _tc2_body(degp_ref, nrc_ref, dinv_ref, q_ref):
    deg = jnp.sum(degp_ref[...], axis=0)[:, None] + 1.0
    dinv = lax.rsqrt(deg)
    dinv_ref[...] = dinv
    q_ref[...] = dinv * nrc_ref[...]


def _tc2(degp, nrc):
    return pl.pallas_call(
        _tc2_body,
        out_shape=(jax.ShapeDtypeStruct((NPAD, 1), f32),
                   jax.ShapeDtypeStruct((NPAD, 1), f32)),
    )(degp, nrc)


def _tc3_body(accp_ref, x_ref, dinv_ref, W1_ref, b1_ref, hn_ref, nr2c_ref):
    dinv = dinv_ref[...][:N]
    acc = accp_ref[0, :N] + accp_ref[1, :N]
    x = x_ref[...]
    pre = dinv * acc + (dinv * dinv) * x
    h = jnp.maximum(jnp.dot(pre, W1_ref[...],
                            preferred_element_type=f32) + b1_ref[...], 0.0)
    nr2 = jnp.sqrt(jnp.sum(h * h, axis=1, keepdims=True))
    nr2c = jnp.maximum(nr2, 1e-8)
    hn_ref[...] = h / nr2c
    nr2c_ref[...] = jnp.concatenate(
        [nr2c, jnp.ones((NPAD - N, 1), f32)], axis=0)


def _tc3(accp, x, dinv1, W1, b1):
    return pl.pallas_call(
        _tc3_body,
        out_shape=(jax.ShapeDtypeStruct((N, F), f32),
                   jax.ShapeDtypeStruct((NPAD, 1), f32)),
    )(accp, x, dinv1, W1, b1)


def _tc4_body(degp_ref, nr2c_ref, hn_ref, W2_ref, dinv_ref, z2_ref):
    deg = jnp.sum(degp_ref[...], axis=0)[:, None] + 1.0
    dinv = lax.rsqrt(deg)
    dinv_ref[...] = dinv
    scale = (dinv * nr2c_ref[...])[:N]
    z2_ref[...] = jnp.dot(scale * hn_ref[...], W2_ref[...],
                          preferred_element_type=f32)


def _tc4(degp, nr2c, hn, W2):
    return pl.pallas_call(
        _tc4_body,
        out_shape=(jax.ShapeDtypeStruct((NPAD, 1), f32),
                   jax.ShapeDtypeStruct((N, NCLASS), f32)),
    )(degp, nr2c, hn, W2)


def _tc5_body(accp_ref, z2_ref, dinv_ref, b2_ref, out_ref):
    acc = accp_ref[0, :N] + accp_ref[1, :N] + z2_ref[...]
    out_ref[...] = dinv_ref[...][:N] * acc + b2_ref[...]


def _tc5(accp, z2, dinv2, b2):
    return pl.pallas_call(
        _tc5_body,
        out_shape=jax.ShapeDtypeStruct((N, NCLASS), f32),
    )(accp, z2, dinv2, b2)


def kernel(x, adj, W1, b1, W2, b2):
    E = adj.shape[1]
    pad = EPAD - E
    row = adj[0]
    col = adj[1]
    zpad = jnp.zeros((pad,), i32)
    rowp = jnp.concatenate([row, zpad]).reshape(NW, NCH, C)
    colp = jnp.concatenate([col, zpad]).reshape(NW, NCH, C)

    xn, nrc = _tc1(x)
    val1, deg1p = _sim_pass1(xn, rowp, colp)
    dinv1, q1 = _tc2(deg1p, nrc)
    acc1p = _agg_pass1(xn, rowp, colp, val1, q1.reshape(NPAD))
    hn, nr2c = _tc3(acc1p, x, dinv1, W1, b1)
    val2, deg2p = _sim_pass2(hn, rowp, colp, val1)
    dinv2, z2 = _tc4(deg2p, nr2c, hn, W2)
    acc2p = _agg_pass2(z2, rowp, colp, val2)
    return _tc5(acc2p, z2, dinv2, b2)


# speedup vs baseline: 4.6331x; 1.1844x over previous
import functools

import jax
import jax.numpy as jnp
from jax import lax
from jax.experimental import pallas as pl
from jax.experimental.pallas import tpu as pltpu
from jax.experimental.pallas import tpu_sc as plsc

N = 10000
NPAD = 10240
F = 128
NCLASS = 64
THR = 0.1
NC = 2
NS = 16
NW = NC * NS
C = 128
NCH = 80
EPT = NCH * C
EPAD = NW * EPT
SL = NPAD // NS
f32 = jnp.float32
i32 = jnp.int32

_mesh = plsc.VectorSubcoreMesh(core_axis_name="c", subcore_axis_name="s")


def _make_sim_pass(has_prev):
    out_type = (jax.ShapeDtypeStruct((NW, NCH, C), f32),
                jax.ShapeDtypeStruct((NW, NPAD), f32))
    scratch = [
        pltpu.VMEM((NCH, C), i32),
        pltpu.VMEM((NCH, C), i32),
        pltpu.VMEM((C, F), f32),
        pltpu.VMEM((C, F), f32),
        pltpu.VMEM((C, F), f32),
        pltpu.VMEM((C, F), f32),
        pltpu.VMEM((C,), f32),
        pltpu.VMEM((NPAD,), f32),
        pltpu.SemaphoreType.DMA,
        pltpu.SemaphoreType.DMA,
    ]
    if has_prev:
        scratch.insert(2, pltpu.VMEM((NCH, C), f32))

    def body(*refs):
        if has_prev:
            (feat_hbm, rowp_hbm, colp_hbm, prev_hbm, val_hbm, deg_hbm,
             rowv, colv, prevv, rbuf, cbuf, rbuf2, cbuf2, valb, degl,
             sem1, sem2) = refs
        else:
            (feat_hbm, rowp_hbm, colp_hbm, val_hbm, deg_hbm,
             rowv, colv, rbuf, cbuf, rbuf2, cbuf2, valb, degl,
             sem1, sem2) = refs
        cid = lax.axis_index("c")
        sid = lax.axis_index("s")
        wid = sid * NC + cid
        pltpu.sync_copy(rowp_hbm.at[wid], rowv)
        pltpu.sync_copy(colp_hbm.at[wid], colv)
        if has_prev:
            pltpu.sync_copy(prev_hbm.at[wid], prevv)

        zero16 = jnp.zeros((16,), f32)

        def zbody(i, carry):
            degl[pl.ds(i * 16, 16)] = zero16
            return carry
        lax.fori_loop(0, NPAD // 16, zbody, 0)

        iota = lax.iota(i32, 16)
        zv = jnp.zeros((16,), f32)
        z0 = jnp.zeros((16,), i32)

        def compute(j, rbuf_, cbuf_):
            for g in range(C // 16):
                rows16 = iota + (g * 16)

                def dot(i, carry):
                    a0, a1, a2, a3, kv = carry
                    x0 = plsc.load_gather(rbuf_, [rows16, kv])
                    y0 = plsc.load_gather(cbuf_, [rows16, kv])
                    x1 = plsc.load_gather(rbuf_, [rows16, kv + 1])
                    y1 = plsc.load_gather(cbuf_, [rows16, kv + 1])
                    x2 = plsc.load_gather(rbuf_, [rows16, kv + 2])
                    y2 = plsc.load_gather(cbuf_, [rows16, kv + 2])
                    x3 = plsc.load_gather(rbuf_, [rows16, kv + 3])
                    y3 = plsc.load_gather(cbuf_, [rows16, kv + 3])
                    return (a0 + x0 * y0, a1 + x1 * y1,
                            a2 + x2 * y2, a3 + x3 * y3, kv + 4)
                a0, a1, a2, a3, _ = lax.fori_loop(
                    0, F // 4, dot, (zv, zv, zv, zv, z0), unroll=4)
                sims = (a0 + a1) + (a2 + a3)
                rv = rowv[j, pl.ds(g * 16, 16)]
                cv = colv[j, pl.ds(g * 16, 16)]
                m = (sims >= THR) & (rv != cv)
                if has_prev:
                    m = m & (prevv[j, pl.ds(g * 16, 16)] > 0.0)
                val = jnp.where(m, sims, 0.0)
                valb[pl.ds(g * 16, 16)] = val
                plsc.addupdate_scatter(degl, [rv], val)
            pltpu.sync_copy(valb, val_hbm.at[wid, j])

        def chunk2(jj, carry):
            j = jj * 2

            @pl.when(jj == 0)
            def _():
                pltpu.async_copy(feat_hbm.at[rowv.at[j]], rbuf, sem1)
                pltpu.async_copy(feat_hbm.at[colv.at[j]], cbuf, sem1)
            pltpu.make_async_copy(feat_hbm.at[rowv.at[j]], rbuf, sem1).wait()
            pltpu.make_async_copy(feat_hbm.at[colv.at[j]], cbuf, sem1).wait()
            pltpu.async_copy(feat_hbm.at[rowv.at[j + 1]], rbuf2, sem2)
            pltpu.async_copy(feat_hbm.at[colv.at[j + 1]], cbuf2, sem2)
            compute(j, rbuf, cbuf)
            pltpu.make_async_copy(
                feat_hbm.at[rowv.at[j + 1]], rbuf2, sem2).wait()
            pltpu.make_async_copy(
                feat_hbm.at[colv.at[j + 1]], cbuf2, sem2).wait()

            @pl.when(jj < NCH // 2 - 1)
            def _():
                pltpu.async_copy(feat_hbm.at[rowv.at[j + 2]], rbuf, sem1)
                pltpu.async_copy(feat_hbm.at[colv.at[j + 2]], cbuf, sem1)
            compute(j + 1, rbuf2, cbuf2)
            return carry
        lax.fori_loop(0, NCH // 2, chunk2, 0)
        pltpu.sync_copy(degl, deg_hbm.at[wid])

    return pl.kernel(body, out_type=out_type, mesh=_mesh,
                     compiler_params=pltpu.CompilerParams(
                         needs_layout_passes=False),
                     scratch_types=scratch)


def _make_agg_pass(D, use_q):
    out_type = jax.ShapeDtypeStruct((NC, NPAD, D), f32)
    scratch = [
        pltpu.VMEM((C,), i32),
        pltpu.VMEM((C,), i32),
        pltpu.VMEM((C,), f32),
        pltpu.VMEM((C, D), f32),
        pltpu.VMEM((C,), f32),
        pltpu.VMEM((16, D), f32),
        pltpu.VMEM_SHARED((NPAD, D), f32),
        pltpu.SemaphoreType.DMA,
        pltpu.SemaphoreType.DMA,
    ]
    if use_q:
        scratch.insert(3, pltpu.VMEM((NPAD,), f32))

    def body(*refs):
        if use_q:
            (feat_hbm, rowp_hbm, colp_hbm, val_hbm, q_hbm, acc_hbm,
             rowc, colc, valc, qv, gbuf, wbuf, zbuf, acc_sh,
             sem1, sem2) = refs
        else:
            (feat_hbm, rowp_hbm, colp_hbm, val_hbm, acc_hbm,
             rowc, colc, valc, gbuf, wbuf, zbuf, acc_sh, sem1, sem2) = refs
        cid = lax.axis_index("c")
        sid = lax.axis_index("s")
        wid = sid * NC + cid
        if use_q:
            pltpu.sync_copy(q_hbm, qv)

        zero16 = jnp.zeros((16,), f32)

        def zrow(r, carry):
            for k in range(D // 16):
                zbuf[r, pl.ds(k * 16, 16)] = zero16
            return carry
        lax.fori_loop(0, 16, zrow, 0)

        def zacc(t, carry):
            pltpu.sync_copy(zbuf, acc_sh.at[pl.ds(sid * SL + t * 16, 16)])
            return carry
        lax.fori_loop(0, SL // 16, zacc, 0)
        plsc.subcore_barrier()

        def chunk(j, carry):
            pltpu.sync_copy(rowp_hbm.at[wid, j], rowc)
            pltpu.sync_copy(colp_hbm.at[wid, j], colc)
            pltpu.sync_copy(val_hbm.at[wid, j], valc)
            pltpu.async_copy(feat_hbm.at[colc], gbuf, sem1).wait()
            for g in range(C // 16):
                val = valc[pl.ds(g * 16, 16)]
                if use_q:
                    cv = colc[pl.ds(g * 16, 16)]
                    val = val * plsc.load_gather(qv, [cv])
                wbuf[pl.ds(g * 16, 16)] = val

            def scale(e, c2):
                wv = plsc.load_gather(wbuf, [jnp.full((16,), e, i32)])
                for k in range(D // 16):
                    gbuf[e, pl.ds(k * 16, 16)] = (
                        gbuf[e, pl.ds(k * 16, 16)] * wv)
                return c2
            lax.fori_loop(0, C, scale, 0)
            pltpu.sync_copy(gbuf, acc_sh.at[rowc], add=True)
            return carry
        lax.fori_loop(0, NCH, chunk, 0)
        plsc.subcore_barrier()

        def dump(t, carry):
            pltpu.sync_copy(acc_sh.at[pl.ds(sid * SL + t * 16, 16)], zbuf)
            pltpu.sync_copy(zbuf, acc_hbm.at[cid, pl.ds(sid * SL + t * 16, 16)])
            return carry
        lax.fori_loop(0, SL // 16, dump, 0)

    return pl.kernel(body, out_type=out_type, mesh=_mesh,
                     compiler_params=pltpu.CompilerParams(
                         needs_layout_passes=False,
                         use_tc_tiling_on_sc=(D % 128 == 0)),
                     scratch_types=scratch)


_sim_pass1 = _make_sim_pass(False)
_sim_pass2 = _make_sim_pass(True)
_agg_pass1 = _make_agg_pass(F, True)
_agg_pass2 = _make_agg_pass(NCLASS, False)


def _tc1_body(x_ref, xn_ref, nrc_ref):
    x = x_ref[...]
    nr = jnp.sqrt(jnp.sum(x * x, axis=1, keepdims=True))
    nrc = jnp.maximum(nr, 1e-8)
    xn_ref[...] = x / nrc
    nrc_ref[...] = jnp.concatenate(
        [nrc, jnp.ones((NPAD - N, 1), f32)], axis=0)


def _tc1(x):
    return pl.pallas_call(
        _tc1_body,
        out_shape=(jax.ShapeDtypeStruct((N, F), f32),
                   jax.ShapeDtypeStruct((NPAD, 1), f32)),
    )(x)


def _tc2_body(degp_ref, nrc_ref, dinv_ref, q_ref):
    deg = jnp.sum(degp_ref[...], axis=0)[:, None] + 1.0
    dinv = lax.rsqrt(deg)
    dinv_ref[...] = dinv
    q_ref[...] = dinv * nrc_ref[...]


def _tc2(degp, nrc):
    return pl.pallas_call(
        _tc2_body,
        out_shape=(jax.ShapeDtypeStruct((NPAD, 1), f32),
                   jax.ShapeDtypeStruct((NPAD, 1), f32)),
    )(degp, nrc)


def _tc3_body(accp_ref, x_ref, dinv_ref, W1_ref, b1_ref, hn_ref, nr2c_ref):
    dinv = dinv_ref[...][:N]
    acc = accp_ref[0, :N] + accp_ref[1, :N]
    x = x_ref[...]
    pre = dinv * acc + (dinv * dinv) * x
    h = jnp.maximum(jnp.dot(pre, W1_ref[...],
                            preferred_element_type=f32) + b1_ref[...], 0.0)
    nr2 = jnp.sqrt(jnp.sum(h * h, axis=1, keepdims=True))
    nr2c = jnp.maximum(nr2, 1e-8)
    hn_ref[...] = h / nr2c
    nr2c_ref[...] = jnp.concatenate(
        [nr2c, jnp.ones((NPAD - N, 1), f32)], axis=0)


def _tc3(accp, x, dinv1, W1, b1):
    return pl.pallas_call(
        _tc3_body,
        out_shape=(jax.ShapeDtypeStruct((N, F), f32),
                   jax.ShapeDtypeStruct((NPAD, 1), f32)),
    )(accp, x, dinv1, W1, b1)


def _tc4_body(degp_ref, nr2c_ref, hn_ref, W2_ref, dinv_ref, z2_ref):
    deg = jnp.sum(degp_ref[...], axis=0)[:, None] + 1.0
    dinv = lax.rsqrt(deg)
    dinv_ref[...] = dinv
    scale = (dinv * nr2c_ref[...])[:N]
    z2_ref[...] = jnp.dot(scale * hn_ref[...], W2_ref[...],
                          preferred_element_type=f32)


def _tc4(degp, nr2c, hn, W2):
    return pl.pallas_call(
        _tc4_body,
        out_shape=(jax.ShapeDtypeStruct((NPAD, 1), f32),
                   jax.ShapeDtypeStruct((N, NCLASS), f32)),
    )(degp, nr2c, hn, W2)


def _tc5_body(accp_ref, z2_ref, dinv_ref, b2_ref, out_ref):
    acc = accp_ref[0, :N] + accp_ref[1, :N] + z2_ref[...]
    out_ref[...] = dinv_ref[...][:N] * acc + b2_ref[...]


def _tc5(accp, z2, dinv2, b2):
    return pl.pallas_call(
        _tc5_body,
        out_shape=jax.ShapeDtypeStruct((N, NCLASS), f32),
    )(accp, z2, dinv2, b2)


def kernel(x, adj, W1, b1, W2, b2):
    E = adj.shape[1]
    pad = EPAD - E
    row = adj[0]
    col = adj[1]
    zpad = jnp.zeros((pad,), i32)
    rowp = jnp.concatenate([row, zpad]).reshape(NW, NCH, C)
    colp = jnp.concatenate([col, zpad]).reshape(NW, NCH, C)

    xn, nrc = _tc1(x)
    val1, deg1p = _sim_pass1(xn, rowp, colp)
    dinv1, q1 = _tc2(deg1p, nrc)
    acc1p = _agg_pass1(xn, rowp, colp, val1, q1.reshape(NPAD))
    hn, nr2c = _tc3(acc1p, x, dinv1, W1, b1)
    val2, deg2p = _sim_pass2(hn, rowp, colp, val1)
    dinv2, z2 = _tc4(deg2p, nr2c, hn, W2)
    acc2p = _agg_pass2(z2, rowp, colp, val2)
    return _tc5(acc2p, z2, dinv2, b2)


# speedup vs baseline: 4.7458x; 1.0243x over previous
import functools

import jax
import jax.numpy as jnp
from jax import lax
from jax.experimental import pallas as pl
from jax.experimental.pallas import tpu as pltpu
from jax.experimental.pallas import tpu_sc as plsc

N = 10000
NPAD = 10240
F = 128
NCLASS = 64
THR = 0.1
NC = 2
NS = 16
NW = NC * NS
C = 128
NCH = 80
EPT = NCH * C
EPAD = NW * EPT
SL = NPAD // NS
f32 = jnp.float32
i32 = jnp.int32

_mesh = plsc.VectorSubcoreMesh(core_axis_name="c", subcore_axis_name="s")


def _make_sim_pass(has_prev):
    out_type = (jax.ShapeDtypeStruct((NW, NCH, C), f32),
                jax.ShapeDtypeStruct((NW, NPAD), f32))
    scratch = [
        pltpu.VMEM((NCH, C), i32),
        pltpu.VMEM((NCH, C), i32),
        pltpu.VMEM((C, F), f32),
        pltpu.VMEM((C, F), f32),
        pltpu.VMEM((C, F), f32),
        pltpu.VMEM((C, F), f32),
        pltpu.VMEM((C,), f32),
        pltpu.VMEM((NPAD,), f32),
        pltpu.SemaphoreType.DMA,
        pltpu.SemaphoreType.DMA,
    ]
    if has_prev:
        scratch.insert(2, pltpu.VMEM((NCH, C), f32))

    def body(*refs):
        if has_prev:
            (feat_hbm, rowp_hbm, colp_hbm, prev_hbm, val_hbm, deg_hbm,
             rowv, colv, prevv, rbuf, cbuf, rbuf2, cbuf2, valb, degl,
             sem1, sem2) = refs
        else:
            (feat_hbm, rowp_hbm, colp_hbm, val_hbm, deg_hbm,
             rowv, colv, rbuf, cbuf, rbuf2, cbuf2, valb, degl,
             sem1, sem2) = refs
        cid = lax.axis_index("c")
        sid = lax.axis_index("s")
        wid = sid * NC + cid
        pltpu.sync_copy(rowp_hbm.at[wid], rowv)
        pltpu.sync_copy(colp_hbm.at[wid], colv)
        if has_prev:
            pltpu.sync_copy(prev_hbm.at[wid], prevv)

        zero16 = jnp.zeros((16,), f32)

        def zbody(i, carry):
            degl[pl.ds(i * 16, 16)] = zero16
            return carry
        lax.fori_loop(0, NPAD // 16, zbody, 0)

        iota = lax.iota(i32, 16)
        zv = jnp.zeros((16,), f32)
        z0 = jnp.zeros((16,), i32)

        def compute(j, rbuf_, cbuf_):
            for g in range(C // 16):
                rows16 = iota + (g * 16)

                def dot(i, accs):
                    a0, a1, a2, a3 = accs
                    kv = jnp.full((16,), i, i32)
                    x0 = plsc.load_gather(rbuf_, [rows16, kv])
                    y0 = plsc.load_gather(cbuf_, [rows16, kv])
                    x1 = plsc.load_gather(rbuf_, [rows16, kv + 1])
                    y1 = plsc.load_gather(cbuf_, [rows16, kv + 1])
                    x2 = plsc.load_gather(rbuf_, [rows16, kv + 2])
                    y2 = plsc.load_gather(cbuf_, [rows16, kv + 2])
                    x3 = plsc.load_gather(rbuf_, [rows16, kv + 3])
                    y3 = plsc.load_gather(cbuf_, [rows16, kv + 3])
                    return (a0 + x0 * y0, a1 + x1 * y1,
                            a2 + x2 * y2, a3 + x3 * y3)
                a0, a1, a2, a3 = plsc.parallel_loop(
                    0, F, 4, unroll=4, carry=(zv, zv, zv, zv))(dot)
                sims = (a0 + a1) + (a2 + a3)
                rv = rowv[j, pl.ds(g * 16, 16)]
                cv = colv[j, pl.ds(g * 16, 16)]
                m = (sims >= THR) & (rv != cv)
                if has_prev:
                    m = m & (prevv[j, pl.ds(g * 16, 16)] > 0.0)
                val = jnp.where(m, sims, 0.0)
                valb[pl.ds(g * 16, 16)] = val
                plsc.addupdate_scatter(degl, [rv], val)
            pltpu.sync_copy(valb, val_hbm.at[wid, j])

        def chunk2(jj, carry):
            j = jj * 2

            @pl.when(jj == 0)
            def _():
                pltpu.async_copy(feat_hbm.at[rowv.at[j]], rbuf, sem1)
                pltpu.async_copy(feat_hbm.at[colv.at[j]], cbuf, sem1)
            pltpu.make_async_copy(feat_hbm.at[rowv.at[j]], rbuf, sem1).wait()
            pltpu.make_async_copy(feat_hbm.at[colv.at[j]], cbuf, sem1).wait()
            pltpu.async_copy(feat_hbm.at[rowv.at[j + 1]], rbuf2, sem2)
            pltpu.async_copy(feat_hbm.at[colv.at[j + 1]], cbuf2, sem2)
            compute(j, rbuf, cbuf)
            pltpu.make_async_copy(
                feat_hbm.at[rowv.at[j + 1]], rbuf2, sem2).wait()
            pltpu.make_async_copy(
                feat_hbm.at[colv.at[j + 1]], cbuf2, sem2).wait()

            @pl.when(jj < NCH // 2 - 1)
            def _():
                pltpu.async_copy(feat_hbm.at[rowv.at[j + 2]], rbuf, sem1)
                pltpu.async_copy(feat_hbm.at[colv.at[j + 2]], cbuf, sem1)
            compute(j + 1, rbuf2, cbuf2)
            return carry
        lax.fori_loop(0, NCH // 2, chunk2, 0)
        pltpu.sync_copy(degl, deg_hbm.at[wid])

    return pl.kernel(body, out_type=out_type, mesh=_mesh,
                     compiler_params=pltpu.CompilerParams(
                         needs_layout_passes=False),
                     scratch_types=scratch)


def _make_agg_pass(D, use_q):
    out_type = jax.ShapeDtypeStruct((NC, NPAD, D), f32)
    scratch = [
        pltpu.VMEM((C,), i32),
        pltpu.VMEM((C,), i32),
        pltpu.VMEM((C,), f32),
        pltpu.VMEM((C, D), f32),
        pltpu.VMEM((C,), f32),
        pltpu.VMEM((16, D), f32),
        pltpu.VMEM_SHARED((NPAD, D), f32),
        pltpu.SemaphoreType.DMA,
        pltpu.SemaphoreType.DMA,
    ]
    if use_q:
        scratch.insert(3, pltpu.VMEM((NPAD,), f32))

    def body(*refs):
        if use_q:
            (feat_hbm, rowp_hbm, colp_hbm, val_hbm, q_hbm, acc_hbm,
             rowc, colc, valc, qv, gbuf, wbuf, zbuf, acc_sh,
             sem1, sem2) = refs
        else:
            (feat_hbm, rowp_hbm, colp_hbm, val_hbm, acc_hbm,
             rowc, colc, valc, gbuf, wbuf, zbuf, acc_sh, sem1, sem2) = refs
        cid = lax.axis_index("c")
        sid = lax.axis_index("s")
        wid = sid * NC + cid
        if use_q:
            pltpu.sync_copy(q_hbm, qv)

        zero16 = jnp.zeros((16,), f32)

        def zrow(r, carry):
            for k in range(D // 16):
                zbuf[r, pl.ds(k * 16, 16)] = zero16
            return carry
        lax.fori_loop(0, 16, zrow, 0)

        def zacc(t, carry):
            pltpu.sync_copy(zbuf, acc_sh.at[pl.ds(sid * SL + t * 16, 16)])
            return carry
        lax.fori_loop(0, SL // 16, zacc, 0)
        plsc.subcore_barrier()

        def chunk(j, carry):
            pltpu.sync_copy(rowp_hbm.at[wid, j], rowc)
            pltpu.sync_copy(colp_hbm.at[wid, j], colc)
            pltpu.sync_copy(val_hbm.at[wid, j], valc)
            pltpu.async_copy(feat_hbm.at[colc], gbuf, sem1).wait()
            for g in range(C // 16):
                val = valc[pl.ds(g * 16, 16)]
                if use_q:
                    cv = colc[pl.ds(g * 16, 16)]
                    val = val * plsc.load_gather(qv, [cv])
                wbuf[pl.ds(g * 16, 16)] = val

            @plsc.parallel_loop(0, C, 1, unroll=4)
            def scale(e):
                wv = plsc.load_gather(wbuf, [jnp.full((16,), e, i32)])
                for k in range(D // 16):
                    gbuf[e, pl.ds(k * 16, 16)] = (
                        gbuf[e, pl.ds(k * 16, 16)] * wv)
            pltpu.sync_copy(gbuf, acc_sh.at[rowc], add=True)
            return carry
        lax.fori_loop(0, NCH, chunk, 0)
        plsc.subcore_barrier()

        def dump(t, carry):
            pltpu.sync_copy(acc_sh.at[pl.ds(sid * SL + t * 16, 16)], zbuf)
            pltpu.sync_copy(zbuf, acc_hbm.at[cid, pl.ds(sid * SL + t * 16, 16)])
            return carry
        lax.fori_loop(0, SL // 16, dump, 0)

    return pl.kernel(body, out_type=out_type, mesh=_mesh,
                     compiler_params=pltpu.CompilerParams(
                         needs_layout_passes=False,
                         use_tc_tiling_on_sc=(D % 128 == 0)),
                     scratch_types=scratch)


_sim_pass1 = _make_sim_pass(False)
_sim_pass2 = _make_sim_pass(True)
_agg_pass1 = _make_agg_pass(F, True)
_agg_pass2 = _make_agg_pass(NCLASS, False)


def _tc1_body(x_ref, xn_ref, nrc_ref):
    x = x_ref[...]
    nr = jnp.sqrt(jnp.sum(x * x, axis=1, keepdims=True))
    nrc = jnp.maximum(nr, 1e-8)
    xn_ref[...] = x / nrc
    nrc_ref[...] = jnp.concatenate(
        [nrc, jnp.ones((NPAD - N, 1), f32)], axis=0)


def _tc1(x):
    return pl.pallas_call(
        _tc1_body,
        out_shape=(jax.ShapeDtypeStruct((N, F), f32),
                   jax.ShapeDtypeStruct((NPAD, 1), f32)),
    )(x)


def _tc2_body(degp_ref, nrc_ref, dinv_ref, q_ref):
    deg = jnp.sum(degp_ref[...], axis=0)[:, None] + 1.0
    dinv = lax.rsqrt(deg)
    dinv_ref[...] = dinv
    q_ref[...] = dinv * nrc_ref[...]


def _tc2(degp, nrc):
    return pl.pallas_call(
        _tc2_body,
        out_shape=(jax.ShapeDtypeStruct((NPAD, 1), f32),
                   jax.ShapeDtypeStruct((NPAD, 1), f32)),
    )(degp, nrc)


def _tc3_body(accp_ref, x_ref, dinv_ref, W1_ref, b1_ref, hn_ref, nr2c_ref):
    dinv = dinv_ref[...][:N]
    acc = accp_ref[0, :N] + accp_ref[1, :N]
    x = x_ref[...]
    pre = dinv * acc + (dinv * dinv) * x
    h = jnp.maximum(jnp.dot(pre, W1_ref[...],
                            preferred_element_type=f32) + b1_ref[...], 0.0)
    nr2 = jnp.sqrt(jnp.sum(h * h, axis=1, keepdims=True))
    nr2c = jnp.maximum(nr2, 1e-8)
    hn_ref[...] = h / nr2c
    nr2c_ref[...] = jnp.concatenate(
        [nr2c, jnp.ones((NPAD - N, 1), f32)], axis=0)


def _tc3(accp, x, dinv1, W1, b1):
    return pl.pallas_call(
        _tc3_body,
        out_shape=(jax.ShapeDtypeStruct((N, F), f32),
                   jax.ShapeDtypeStruct((NPAD, 1), f32)),
    )(accp, x, dinv1, W1, b1)


def _tc4_body(degp_ref, nr2c_ref, hn_ref, W2_ref, dinv_ref, z2_ref):
    deg = jnp.sum(degp_ref[...], axis=0)[:, None] + 1.0
    dinv = lax.rsqrt(deg)
    dinv_ref[...] = dinv
    scale = (dinv * nr2c_ref[...])[:N]
    z2_ref[...] = jnp.dot(scale * hn_ref[...], W2_ref[...],
                          preferred_element_type=f32)


def _tc4(degp, nr2c, hn, W2):
    return pl.pallas_call(
        _tc4_body,
        out_shape=(jax.ShapeDtypeStruct((NPAD, 1), f32),
                   jax.ShapeDtypeStruct((N, NCLASS), f32)),
    )(degp, nr2c, hn, W2)


def _tc5_body(accp_ref, z2_ref, dinv_ref, b2_ref, out_ref):
    acc = accp_ref[0, :N] + accp_ref[1, :N] + z2_ref[...]
    out_ref[...] = dinv_ref[...][:N] * acc + b2_ref[...]


def _tc5(accp, z2, dinv2, b2):
    return pl.pallas_call(
        _tc5_body,
        out_shape=jax.ShapeDtypeStruct((N, NCLASS), f32),
    )(accp, z2, dinv2, b2)


def kernel(x, adj, W1, b1, W2, b2):
    E = adj.shape[1]
    pad = EPAD - E
    row = adj[0]
    col = adj[1]
    zpad = jnp.zeros((pad,), i32)
    rowp = jnp.concatenate([row, zpad]).reshape(NW, NCH, C)
    colp = jnp.concatenate([col, zpad]).reshape(NW, NCH, C)

    xn, nrc = _tc1(x)
    val1, deg1p = _sim_pass1(xn, rowp, colp)
    dinv1, q1 = _tc2(deg1p, nrc)
    acc1p = _agg_pass1(xn, rowp, colp, val1, q1.reshape(NPAD))
    hn, nr2c = _tc3(acc1p, x, dinv1, W1, b1)
    val2, deg2p = _sim_pass2(hn, rowp, colp, val1)
    dinv2, z2 = _tc4(deg2p, nr2c, hn, W2)
    acc2p = _agg_pass2(z2, rowp, colp, val2)
    return _tc5(acc2p, z2, dinv2, b2)


# speedup vs baseline: 16.8138x; 3.5428x over previous
import functools

import jax
import jax.numpy as jnp
from jax import lax
from jax.experimental import pallas as pl
from jax.experimental.pallas import tpu as pltpu
from jax.experimental.pallas import tpu_sc as plsc

N = 10000
NPAD = 10240
F = 128
NCLASS = 64
THR = 0.1
NC = 2
NS = 16
NW = NC * NS
C = 128
NCH = 80
EPT = NCH * C
EPAD = NW * EPT
SL = NPAD // NS
f32 = jnp.float32
i32 = jnp.int32

_mesh = plsc.VectorSubcoreMesh(core_axis_name="c", subcore_axis_name="s")


def _make_sim_pass(has_prev):
    out_type = (jax.ShapeDtypeStruct((NW, NCH, C), f32),
                jax.ShapeDtypeStruct((NW, NPAD), f32))
    scratch = [
        pltpu.VMEM((NCH, C), i32),
        pltpu.VMEM((NCH, C), i32),
        pltpu.VMEM((C, F), f32),
        pltpu.VMEM((C, F), f32),
        pltpu.VMEM((C, F), f32),
        pltpu.VMEM((C, F), f32),
        pltpu.VMEM((C,), f32),
        pltpu.VMEM((NPAD,), f32),
        pltpu.SemaphoreType.DMA,
        pltpu.SemaphoreType.DMA,
    ]
    if has_prev:
        scratch.insert(2, pltpu.VMEM((NCH, C), f32))

    def body(*refs):
        if has_prev:
            (feat_hbm, rowp_hbm, colp_hbm, prev_hbm, val_hbm, deg_hbm,
             rowv, colv, prevv, rbuf, cbuf, rbuf2, cbuf2, valb, degl,
             sem1, sem2) = refs
        else:
            (feat_hbm, rowp_hbm, colp_hbm, val_hbm, deg_hbm,
             rowv, colv, rbuf, cbuf, rbuf2, cbuf2, valb, degl,
             sem1, sem2) = refs
        cid = lax.axis_index("c")
        sid = lax.axis_index("s")
        wid = sid * NC + cid
        pltpu.sync_copy(rowp_hbm.at[wid], rowv)
        pltpu.sync_copy(colp_hbm.at[wid], colv)
        if has_prev:
            pltpu.sync_copy(prev_hbm.at[wid], prevv)

        zero16 = jnp.zeros((16,), f32)

        def zbody(i, carry):
            degl[pl.ds(i * 16, 16)] = zero16
            return carry
        lax.fori_loop(0, NPAD // 16, zbody, 0)

        iota = lax.iota(i32, 16)
        zv = jnp.zeros((16,), f32)
        z0 = jnp.zeros((16,), i32)

        def compute(j, rbuf_, cbuf_):
            for g in range(C // 16):
                rows16 = iota + (g * 16)

                def dot(i, accs):
                    a0, a1, a2, a3 = accs
                    kv = jnp.full((16,), i, i32) + iota
                    k0 = kv & (F - 1)
                    k1 = (kv + 1) & (F - 1)
                    k2 = (kv + 2) & (F - 1)
                    k3 = (kv + 3) & (F - 1)
                    x0 = plsc.load_gather(rbuf_, [rows16, k0])
                    y0 = plsc.load_gather(cbuf_, [rows16, k0])
                    x1 = plsc.load_gather(rbuf_, [rows16, k1])
                    y1 = plsc.load_gather(cbuf_, [rows16, k1])
                    x2 = plsc.load_gather(rbuf_, [rows16, k2])
                    y2 = plsc.load_gather(cbuf_, [rows16, k2])
                    x3 = plsc.load_gather(rbuf_, [rows16, k3])
                    y3 = plsc.load_gather(cbuf_, [rows16, k3])
                    return (a0 + x0 * y0, a1 + x1 * y1,
                            a2 + x2 * y2, a3 + x3 * y3)
                a0, a1, a2, a3 = plsc.parallel_loop(
                    0, F, 4, unroll=4, carry=(zv, zv, zv, zv))(dot)
                sims = (a0 + a1) + (a2 + a3)
                rv = rowv[j, pl.ds(g * 16, 16)]
                cv = colv[j, pl.ds(g * 16, 16)]
                m = (sims >= THR) & (rv != cv)
                if has_prev:
                    m = m & (prevv[j, pl.ds(g * 16, 16)] > 0.0)
                val = jnp.where(m, sims, 0.0)
                valb[pl.ds(g * 16, 16)] = val
                plsc.addupdate_scatter(degl, [rv], val)
            pltpu.sync_copy(valb, val_hbm.at[wid, j])

        def chunk2(jj, carry):
            j = jj * 2

            @pl.when(jj == 0)
            def _():
                pltpu.async_copy(feat_hbm.at[rowv.at[j]], rbuf, sem1)
                pltpu.async_copy(feat_hbm.at[colv.at[j]], cbuf, sem1)
            pltpu.make_async_copy(feat_hbm.at[rowv.at[j]], rbuf, sem1).wait()
            pltpu.make_async_copy(feat_hbm.at[colv.at[j]], cbuf, sem1).wait()
            pltpu.async_copy(feat_hbm.at[rowv.at[j + 1]], rbuf2, sem2)
            pltpu.async_copy(feat_hbm.at[colv.at[j + 1]], cbuf2, sem2)
            compute(j, rbuf, cbuf)
            pltpu.make_async_copy(
                feat_hbm.at[rowv.at[j + 1]], rbuf2, sem2).wait()
            pltpu.make_async_copy(
                feat_hbm.at[colv.at[j + 1]], cbuf2, sem2).wait()

            @pl.when(jj < NCH // 2 - 1)
            def _():
                pltpu.async_copy(feat_hbm.at[rowv.at[j + 2]], rbuf, sem1)
                pltpu.async_copy(feat_hbm.at[colv.at[j + 2]], cbuf, sem1)
            compute(j + 1, rbuf2, cbuf2)
            return carry
        lax.fori_loop(0, NCH // 2, chunk2, 0)
        pltpu.sync_copy(degl, deg_hbm.at[wid])

    return pl.kernel(body, out_type=out_type, mesh=_mesh,
                     compiler_params=pltpu.CompilerParams(
                         needs_layout_passes=False),
                     scratch_types=scratch)


def _make_agg_pass(D, use_q):
    out_type = jax.ShapeDtypeStruct((NC, NPAD, D), f32)
    scratch = [
        pltpu.VMEM((C,), i32),
        pltpu.VMEM((C,), i32),
        pltpu.VMEM((C,), f32),
        pltpu.VMEM((C, D), f32),
        pltpu.VMEM((C,), f32),
        pltpu.VMEM((16, D), f32),
        pltpu.VMEM_SHARED((NPAD, D), f32),
        pltpu.SemaphoreType.DMA,
        pltpu.SemaphoreType.DMA,
    ]
    if use_q:
        scratch.insert(3, pltpu.VMEM((NPAD,), f32))

    def body(*refs):
        if use_q:
            (feat_hbm, rowp_hbm, colp_hbm, val_hbm, q_hbm, acc_hbm,
             rowc, colc, valc, qv, gbuf, wbuf, zbuf, acc_sh,
             sem1, sem2) = refs
        else:
            (feat_hbm, rowp_hbm, colp_hbm, val_hbm, acc_hbm,
             rowc, colc, valc, gbuf, wbuf, zbuf, acc_sh, sem1, sem2) = refs
        cid = lax.axis_index("c")
        sid = lax.axis_index("s")
        wid = sid * NC + cid
        if use_q:
            pltpu.sync_copy(q_hbm, qv)

        zero16 = jnp.zeros((16,), f32)

        def zrow(r, carry):
            for k in range(D // 16):
                zbuf[r, pl.ds(k * 16, 16)] = zero16
            return carry
        lax.fori_loop(0, 16, zrow, 0)

        def zacc(t, carry):
            pltpu.sync_copy(zbuf, acc_sh.at[pl.ds(sid * SL + t * 16, 16)])
            return carry
        lax.fori_loop(0, SL // 16, zacc, 0)
        plsc.subcore_barrier()

        def chunk(j, carry):
            pltpu.sync_copy(rowp_hbm.at[wid, j], rowc)
            pltpu.sync_copy(colp_hbm.at[wid, j], colc)
            pltpu.sync_copy(val_hbm.at[wid, j], valc)
            pltpu.async_copy(feat_hbm.at[colc], gbuf, sem1).wait()
            for g in range(C // 16):
                val = valc[pl.ds(g * 16, 16)]
                if use_q:
                    cv = colc[pl.ds(g * 16, 16)]
                    val = val * plsc.load_gather(qv, [cv])
                wbuf[pl.ds(g * 16, 16)] = val

            @plsc.parallel_loop(0, C, 1, unroll=4)
            def scale(e):
                wv = plsc.load_gather(wbuf, [jnp.full((16,), e, i32)])
                for k in range(D // 16):
                    gbuf[e, pl.ds(k * 16, 16)] = (
                        gbuf[e, pl.ds(k * 16, 16)] * wv)
            pltpu.sync_copy(gbuf, acc_sh.at[rowc], add=True)
            return carry
        lax.fori_loop(0, NCH, chunk, 0)
        plsc.subcore_barrier()

        def dump(t, carry):
            pltpu.sync_copy(acc_sh.at[pl.ds(sid * SL + t * 16, 16)], zbuf)
            pltpu.sync_copy(zbuf, acc_hbm.at[cid, pl.ds(sid * SL + t * 16, 16)])
            return carry
        lax.fori_loop(0, SL // 16, dump, 0)

    return pl.kernel(body, out_type=out_type, mesh=_mesh,
                     compiler_params=pltpu.CompilerParams(
                         needs_layout_passes=False,
                         use_tc_tiling_on_sc=(D % 128 == 0)),
                     scratch_types=scratch)


_sim_pass1 = _make_sim_pass(False)
_sim_pass2 = _make_sim_pass(True)
_agg_pass1 = _make_agg_pass(F, True)
_agg_pass2 = _make_agg_pass(NCLASS, False)


def _tc1_body(x_ref, xn_ref, nrc_ref):
    x = x_ref[...]
    nr = jnp.sqrt(jnp.sum(x * x, axis=1, keepdims=True))
    nrc = jnp.maximum(nr, 1e-8)
    xn_ref[...] = x / nrc
    nrc_ref[...] = jnp.concatenate(
        [nrc, jnp.ones((NPAD - N, 1), f32)], axis=0)


def _tc1(x):
    return pl.pallas_call(
        _tc1_body,
        out_shape=(jax.ShapeDtypeStruct((N, F), f32),
                   jax.ShapeDtypeStruct((NPAD, 1), f32)),
    )(x)


def _tc2_body(degp_ref, nrc_ref, dinv_ref, q_ref):
    deg = jnp.sum(degp_ref[...], axis=0)[:, None] + 1.0
    dinv = lax.rsqrt(deg)
    dinv_ref[...] = dinv
    q_ref[...] = dinv * nrc_ref[...]


def _tc2(degp, nrc):
    return pl.pallas_call(
        _tc2_body,
        out_shape=(jax.ShapeDtypeStruct((NPAD, 1), f32),
                   jax.ShapeDtypeStruct((NPAD, 1), f32)),
    )(degp, nrc)


def _tc3_body(accp_ref, x_ref, dinv_ref, W1_ref, b1_ref, hn_ref, nr2c_ref):
    dinv = dinv_ref[...][:N]
    acc = accp_ref[0, :N] + accp_ref[1, :N]
    x = x_ref[...]
    pre = dinv * acc + (dinv * dinv) * x
    h = jnp.maximum(jnp.dot(pre, W1_ref[...],
                            preferred_element_type=f32) + b1_ref[...], 0.0)
    nr2 = jnp.sqrt(jnp.sum(h * h, axis=1, keepdims=True))
    nr2c = jnp.maximum(nr2, 1e-8)
    hn_ref[...] = h / nr2c
    nr2c_ref[...] = jnp.concatenate(
        [nr2c, jnp.ones((NPAD - N, 1), f32)], axis=0)


def _tc3(accp, x, dinv1, W1, b1):
    return pl.pallas_call(
        _tc3_body,
        out_shape=(jax.ShapeDtypeStruct((N, F), f32),
                   jax.ShapeDtypeStruct((NPAD, 1), f32)),
    )(accp, x, dinv1, W1, b1)


def _tc4_body(degp_ref, nr2c_ref, hn_ref, W2_ref, dinv_ref, z2_ref):
    deg = jnp.sum(degp_ref[...], axis=0)[:, None] + 1.0
    dinv = lax.rsqrt(deg)
    dinv_ref[...] = dinv
    scale = (dinv * nr2c_ref[...])[:N]
    z2_ref[...] = jnp.dot(scale * hn_ref[...], W2_ref[...],
                          preferred_element_type=f32)


def _tc4(degp, nr2c, hn, W2):
    return pl.pallas_call(
        _tc4_body,
        out_shape=(jax.ShapeDtypeStruct((NPAD, 1), f32),
                   jax.ShapeDtypeStruct((N, NCLASS), f32)),
    )(degp, nr2c, hn, W2)


def _tc5_body(accp_ref, z2_ref, dinv_ref, b2_ref, out_ref):
    acc = accp_ref[0, :N] + accp_ref[1, :N] + z2_ref[...]
    out_ref[...] = dinv_ref[...][:N] * acc + b2_ref[...]


def _tc5(accp, z2, dinv2, b2):
    return pl.pallas_call(
        _tc5_body,
        out_shape=jax.ShapeDtypeStruct((N, NCLASS), f32),
    )(accp, z2, dinv2, b2)


def kernel(x, adj, W1, b1, W2, b2):
    E = adj.shape[1]
    pad = EPAD - E
    row = adj[0]
    col = adj[1]
    zpad = jnp.arange(pad, dtype=i32) % N
    rowp = jnp.concatenate([row, zpad]).reshape(NW, NCH, C)
    colp = jnp.concatenate([col, zpad]).reshape(NW, NCH, C)

    xn, nrc = _tc1(x)
    val1, deg1p = _sim_pass1(xn, rowp, colp)
    dinv1, q1 = _tc2(deg1p, nrc)
    acc1p = _agg_pass1(xn, rowp, colp, val1, q1.reshape(NPAD))
    hn, nr2c = _tc3(acc1p, x, dinv1, W1, b1)
    val2, deg2p = _sim_pass2(hn, rowp, colp, val1)
    dinv2, z2 = _tc4(deg2p, nr2c, hn, W2)
    acc2p = _agg_pass2(z2, rowp, colp, val2)
    return _tc5(acc2p, z2, dinv2, b2)


# speedup vs baseline: 22.9532x; 1.3651x over previous
import functools

import jax
import jax.numpy as jnp
from jax import lax
from jax.experimental import pallas as pl
from jax.experimental.pallas import tpu as pltpu
from jax.experimental.pallas import tpu_sc as plsc

N = 10000
NPAD = 10240
F = 128
NCLASS = 64
THR = 0.1
NC = 2
NS = 16
NW = NC * NS
C = 128
EPT = 10752
NCH = EPT // C
EPAD = NW * EPT
SL = NPAD // NS
f32 = jnp.float32
i32 = jnp.int32

_mesh = plsc.VectorSubcoreMesh(core_axis_name="c", subcore_axis_name="s")


def _make_sim_pass(has_prev):
    out_type = (jax.ShapeDtypeStruct((NW, NCH, C), f32),
                jax.ShapeDtypeStruct((NW, NPAD), f32))
    scratch = [
        pltpu.VMEM((NCH, C), i32),
        pltpu.VMEM((NCH, C), i32),
        pltpu.VMEM((C, F), f32),
        pltpu.VMEM((C, F), f32),
        pltpu.VMEM((C, F), f32),
        pltpu.VMEM((C, F), f32),
        pltpu.VMEM((C,), f32),
        pltpu.VMEM((NPAD,), f32),
        pltpu.SemaphoreType.DMA,
        pltpu.SemaphoreType.DMA,
    ]
    if has_prev:
        scratch.insert(2, pltpu.VMEM((NCH, C), f32))

    def body(*refs):
        if has_prev:
            (feat_hbm, rowp_hbm, colp_hbm, prev_hbm, val_hbm, deg_hbm,
             rowv, colv, prevv, rbuf, cbuf, rbuf2, cbuf2, valb, degl,
             sem1, sem2) = refs
        else:
            (feat_hbm, rowp_hbm, colp_hbm, val_hbm, deg_hbm,
             rowv, colv, rbuf, cbuf, rbuf2, cbuf2, valb, degl,
             sem1, sem2) = refs
        cid = lax.axis_index("c")
        sid = lax.axis_index("s")
        wid = sid * NC + cid
        pltpu.sync_copy(rowp_hbm.at[wid], rowv)
        pltpu.sync_copy(colp_hbm.at[wid], colv)
        if has_prev:
            pltpu.sync_copy(prev_hbm.at[wid], prevv)

        zero16 = jnp.zeros((16,), f32)

        def zbody(i, carry):
            degl[pl.ds(i * 16, 16)] = zero16
            return carry
        lax.fori_loop(0, NPAD // 16, zbody, 0)

        iota = lax.iota(i32, 16)
        zv = jnp.zeros((16,), f32)
        z0 = jnp.zeros((16,), i32)

        def compute(j, rbuf_, cbuf_):
            for g in range(C // 16):
                rows16 = iota + (g * 16)

                def dot(i, accs):
                    a0, a1, a2, a3 = accs
                    kv = jnp.full((16,), i, i32) + iota
                    k0 = kv & (F - 1)
                    k1 = (kv + 1) & (F - 1)
                    k2 = (kv + 2) & (F - 1)
                    k3 = (kv + 3) & (F - 1)
                    x0 = plsc.load_gather(rbuf_, [rows16, k0])
                    y0 = plsc.load_gather(cbuf_, [rows16, k0])
                    x1 = plsc.load_gather(rbuf_, [rows16, k1])
                    y1 = plsc.load_gather(cbuf_, [rows16, k1])
                    x2 = plsc.load_gather(rbuf_, [rows16, k2])
                    y2 = plsc.load_gather(cbuf_, [rows16, k2])
                    x3 = plsc.load_gather(rbuf_, [rows16, k3])
                    y3 = plsc.load_gather(cbuf_, [rows16, k3])
                    return (a0 + x0 * y0, a1 + x1 * y1,
                            a2 + x2 * y2, a3 + x3 * y3)
                a0, a1, a2, a3 = plsc.parallel_loop(
                    0, F, 4, unroll=4, carry=(zv, zv, zv, zv))(dot)
                sims = (a0 + a1) + (a2 + a3)
                rv = rowv[j, pl.ds(g * 16, 16)]
                cv = colv[j, pl.ds(g * 16, 16)]
                m = (sims >= THR) & (rv != cv)
                if has_prev:
                    m = m & (prevv[j, pl.ds(g * 16, 16)] > 0.0)
                val = jnp.where(m, sims, 0.0)
                valb[pl.ds(g * 16, 16)] = val
                plsc.addupdate_scatter(degl, [rv], val)
            pltpu.sync_copy(valb, val_hbm.at[wid, j])

        def chunk2(jj, carry):
            j = jj * 2

            @pl.when(jj == 0)
            def _():
                pltpu.async_copy(feat_hbm.at[rowv.at[j]], rbuf, sem1)
                pltpu.async_copy(feat_hbm.at[colv.at[j]], cbuf, sem1)
            pltpu.make_async_copy(feat_hbm.at[rowv.at[j]], rbuf, sem1).wait()
            pltpu.make_async_copy(feat_hbm.at[colv.at[j]], cbuf, sem1).wait()
            pltpu.async_copy(feat_hbm.at[rowv.at[j + 1]], rbuf2, sem2)
            pltpu.async_copy(feat_hbm.at[colv.at[j + 1]], cbuf2, sem2)
            compute(j, rbuf, cbuf)
            pltpu.make_async_copy(
                feat_hbm.at[rowv.at[j + 1]], rbuf2, sem2).wait()
            pltpu.make_async_copy(
                feat_hbm.at[colv.at[j + 1]], cbuf2, sem2).wait()

            @pl.when(jj < NCH // 2 - 1)
            def _():
                pltpu.async_copy(feat_hbm.at[rowv.at[j + 2]], rbuf, sem1)
                pltpu.async_copy(feat_hbm.at[colv.at[j + 2]], cbuf, sem1)
            compute(j + 1, rbuf2, cbuf2)
            return carry
        lax.fori_loop(0, NCH // 2, chunk2, 0)
        pltpu.sync_copy(degl, deg_hbm.at[wid])

    return pl.kernel(body, out_type=out_type, mesh=_mesh,
                     compiler_params=pltpu.CompilerParams(
                         needs_layout_passes=False),
                     scratch_types=scratch)


def _make_agg_pass(D, CB):
    NCB = EPT // CB
    T = NCB // 3
    out_type = jax.ShapeDtypeStruct((NC, NPAD, D), f32)
    scratch = [
        pltpu.VMEM((3, CB), i32),
        pltpu.VMEM((3, CB), i32),
        pltpu.VMEM((3, CB), i32),
        pltpu.VMEM((CB,), i32),
        pltpu.VMEM((CB,), i32),
        pltpu.VMEM((CB,), i32),
        pltpu.VMEM((CB, D), f32),
        pltpu.VMEM((CB, D), f32),
        pltpu.VMEM((CB, D), f32),
        pltpu.VMEM((CB + 16,), f32),
        pltpu.VMEM_SHARED((NPAD, D), f32),
        pltpu.SemaphoreType.DMA,
        pltpu.SemaphoreType.DMA,
        pltpu.SemaphoreType.DMA,
    ]

    def body(feat_hbm, ivp_hbm, acc_hbm, iv0, iv1, iv2, rowb0, rowb1, rowb2,
             gbuf0, gbuf1, gbuf2, wbuf, acc_sh, semi, semg, sems):
        cid = lax.axis_index("c")
        sid = lax.axis_index("s")
        wid = sid * NC + cid

        zero16 = jnp.zeros((16,), f32)

        def zrow(r, carry):
            for k in range(D // 16):
                gbuf0[r, pl.ds(k * 16, 16)] = zero16
            return carry
        lax.fori_loop(0, CB, zrow, 0)

        def zacc(t, carry):
            pltpu.sync_copy(gbuf0, acc_sh.at[pl.ds(sid * SL + t * CB, CB)])
            return carry
        lax.fori_loop(0, SL // CB, zacc, 0)
        plsc.subcore_barrier()

        def compute_scale(iv, rowb, gbuf):
            for g in range(CB // 16):
                sl16 = pl.ds(g * 16, 16)
                wbuf[sl16] = plsc.bitcast(iv[2, sl16], f32)
                rowb[sl16] = iv[0, sl16]

            @plsc.parallel_loop(0, CB, 1, unroll=4)
            def scale(e):
                w16 = wbuf[pl.ds(e, 16)]
                wv = jnp.full((16,), w16[0], f32)
                for k in range(D // 16):
                    gbuf[e, pl.ds(k * 16, 16)] = (
                        gbuf[e, pl.ds(k * 16, 16)] * wv)

        def wait_iv(j, iv):
            pltpu.make_async_copy(ivp_hbm.at[wid, j], iv, semi).wait()

        def wait_gather(iv, gbuf):
            pltpu.make_async_copy(feat_hbm.at[iv.at[1]], gbuf, semg).wait()

        def wait_scatter(gbuf, rowb):
            pltpu.make_async_copy(gbuf, acc_sh.at[rowb], sems).wait()

        def triple(t, carry):
            j = 3 * t

            @pl.when(t == 0)
            def _():
                pltpu.sync_copy(ivp_hbm.at[wid, 0 * t], iv0)
                pltpu.async_copy(feat_hbm.at[iv0.at[1]], gbuf0, semg)
                pltpu.async_copy(ivp_hbm.at[wid, 0 * t + 1], iv1, semi)

            wait_gather(iv0, gbuf0)
            wait_iv(j + 1, iv1)

            @pl.when(t > 0)
            def _():
                wait_scatter(gbuf1, rowb1)
            pltpu.async_copy(feat_hbm.at[iv1.at[1]], gbuf1, semg)
            compute_scale(iv0, rowb0, gbuf0)
            pltpu.async_copy(gbuf0, acc_sh.at[rowb0], sems, add=True)
            pltpu.async_copy(ivp_hbm.at[wid, j + 2], iv2, semi)

            wait_gather(iv1, gbuf1)
            wait_iv(j + 2, iv2)

            @pl.when(t > 0)
            def _():
                wait_scatter(gbuf2, rowb2)
            pltpu.async_copy(feat_hbm.at[iv2.at[1]], gbuf2, semg)
            compute_scale(iv1, rowb1, gbuf1)
            pltpu.async_copy(gbuf1, acc_sh.at[rowb1], sems, add=True)

            @pl.when(t < T - 1)
            def _():
                pltpu.async_copy(ivp_hbm.at[wid, j + 3], iv0, semi)

            wait_gather(iv2, gbuf2)
            wait_scatter(gbuf0, rowb0)

            @pl.when(t < T - 1)
            def _():
                wait_iv(j + 3, iv0)
                pltpu.async_copy(feat_hbm.at[iv0.at[1]], gbuf0, semg)
                pltpu.async_copy(ivp_hbm.at[wid, j + 4], iv1, semi)
            compute_scale(iv2, rowb2, gbuf2)
            pltpu.async_copy(gbuf2, acc_sh.at[rowb2], sems, add=True)
            return carry
        lax.fori_loop(0, T, triple, 0)
        wait_scatter(gbuf1, rowb1)
        wait_scatter(gbuf2, rowb2)
        plsc.subcore_barrier()

        def dump(t, carry):
            pltpu.sync_copy(acc_sh.at[pl.ds(sid * SL + t * CB, CB)], gbuf0)
            pltpu.sync_copy(gbuf0, acc_hbm.at[cid, pl.ds(sid * SL + t * CB, CB)])
            return carry
        lax.fori_loop(0, SL // CB, dump, 0)

    return pl.kernel(body, out_type=out_type, mesh=_mesh,
                     compiler_params=pltpu.CompilerParams(
                         needs_layout_passes=False,
                         use_tc_tiling_on_sc=(D % 128 == 0)),
                     scratch_types=scratch)


_sim_pass1 = _make_sim_pass(False)
_sim_pass2 = _make_sim_pass(True)
CB1 = 64
CB2 = 128
_agg_pass1 = _make_agg_pass(F, CB1)
_agg_pass2 = _make_agg_pass(NCLASS, CB2)


def _tc1_body(x_ref, xn_ref):
    x = x_ref[...]
    nr = jnp.sqrt(jnp.sum(x * x, axis=1, keepdims=True))
    xn_ref[...] = x / jnp.maximum(nr, 1e-8)


def _tc1(x):
    return pl.pallas_call(
        _tc1_body,
        out_shape=jax.ShapeDtypeStruct((N, F), f32),
    )(x)


def _tc2_body(degp_ref, x_ref, dinv_ref, z1_ref):
    deg = jnp.sum(degp_ref[...], axis=0)[:, None] + 1.0
    dinv = lax.rsqrt(deg)
    dinv_ref[...] = dinv
    z1_ref[...] = dinv[:N] * x_ref[...]


def _tc2(degp, x):
    return pl.pallas_call(
        _tc2_body,
        out_shape=(jax.ShapeDtypeStruct((NPAD, 1), f32),
                   jax.ShapeDtypeStruct((N, F), f32)),
    )(degp, x)


def _tc3_body(accp_ref, z1_ref, dinv_ref, W1_ref, b1_ref, hn_ref, nr2c_ref):
    dinv = dinv_ref[...][:N]
    acc = accp_ref[0, :N] + accp_ref[1, :N] + z1_ref[...]
    pre = dinv * acc
    h = jnp.maximum(jnp.dot(pre, W1_ref[...],
                            preferred_element_type=f32) + b1_ref[...], 0.0)
    nr2 = jnp.sqrt(jnp.sum(h * h, axis=1, keepdims=True))
    nr2c = jnp.maximum(nr2, 1e-8)
    hn_ref[...] = h / nr2c
    nr2c_ref[...] = jnp.concatenate(
        [nr2c, jnp.ones((NPAD - N, 1), f32)], axis=0)


def _tc3(accp, z1, dinv1, W1, b1):
    return pl.pallas_call(
        _tc3_body,
        out_shape=(jax.ShapeDtypeStruct((N, F), f32),
                   jax.ShapeDtypeStruct((NPAD, 1), f32)),
    )(accp, z1, dinv1, W1, b1)


def _tc4_body(degp_ref, nr2c_ref, hn_ref, W2_ref, dinv_ref, z2_ref):
    deg = jnp.sum(degp_ref[...], axis=0)[:, None] + 1.0
    dinv = lax.rsqrt(deg)
    dinv_ref[...] = dinv
    scale = (dinv * nr2c_ref[...])[:N]
    z2_ref[...] = jnp.dot(scale * hn_ref[...], W2_ref[...],
                          preferred_element_type=f32)


def _tc4(degp, nr2c, hn, W2):
    return pl.pallas_call(
        _tc4_body,
        out_shape=(jax.ShapeDtypeStruct((NPAD, 1), f32),
                   jax.ShapeDtypeStruct((N, NCLASS), f32)),
    )(degp, nr2c, hn, W2)


def _tc5_body(accp_ref, z2_ref, dinv_ref, b2_ref, out_ref):
    acc = accp_ref[0, :N] + accp_ref[1, :N] + z2_ref[...]
    out_ref[...] = dinv_ref[...][:N] * acc + b2_ref[...]


def _tc5(accp, z2, dinv2, b2):
    return pl.pallas_call(
        _tc5_body,
        out_shape=jax.ShapeDtypeStruct((N, NCLASS), f32),
    )(accp, z2, dinv2, b2)


def _make_ivp(rowf, colf, val, CB):
    NCB = EPT // CB
    r = rowf.reshape(NW, NCB, 1, CB)
    c = colf.reshape(NW, NCB, 1, CB)
    v = lax.bitcast_convert_type(val, i32).reshape(NW, NCB, 1, CB)
    return jnp.concatenate([r, c, v], axis=2)


def kernel(x, adj, W1, b1, W2, b2):
    E = adj.shape[1]
    pad = EPAD - E
    row = adj[0]
    col = adj[1]
    zpad = jnp.arange(pad, dtype=i32) % N
    rowf = jnp.concatenate([row, zpad])
    colf = jnp.concatenate([col, zpad])
    rowp = rowf.reshape(NW, NCH, C)
    colp = colf.reshape(NW, NCH, C)

    xn = _tc1(x)
    val1, deg1p = _sim_pass1(xn, rowp, colp)
    dinv1, z1 = _tc2(deg1p, x)
    acc1p = _agg_pass1(z1, _make_ivp(rowf, colf, val1, CB1))
    hn, nr2c = _tc3(acc1p, z1, dinv1, W1, b1)
    val2, deg2p = _sim_pass2(hn, rowp, colp, val1)
    dinv2, z2 = _tc4(deg2p, nr2c, hn, W2)
    acc2p = _agg_pass2(z2, _make_ivp(rowf, colf, val2, CB2))
    return _tc5(acc2p, z2, dinv2, b2)


# speedup vs baseline: 23.1893x; 1.0103x over previous
import functools

import jax
import jax.numpy as jnp
from jax import lax
from jax.experimental import pallas as pl
from jax.experimental.pallas import tpu as pltpu
from jax.experimental.pallas import tpu_sc as plsc

N = 10000
NPAD = 10240
F = 128
NCLASS = 64
THR = 0.1
NC = 2
NS = 16
NW = NC * NS
C = 128
EPT = 10752
NCH = EPT // C
EPAD = NW * EPT
SL = NPAD // NS
f32 = jnp.float32
i32 = jnp.int32

_mesh = plsc.VectorSubcoreMesh(core_axis_name="c", subcore_axis_name="s")


def _make_sim_pass(has_prev):
    out_type = (jax.ShapeDtypeStruct((NW, NCH, C), f32),
                jax.ShapeDtypeStruct((NW, NPAD), f32))
    scratch = [
        pltpu.VMEM((NCH, C), i32),
        pltpu.VMEM((NCH, C), i32),
        pltpu.VMEM((C, F), f32),
        pltpu.VMEM((C, F), f32),
        pltpu.VMEM((C, F), f32),
        pltpu.VMEM((C, F), f32),
        pltpu.VMEM((C,), f32),
        pltpu.VMEM((NPAD,), f32),
        pltpu.SemaphoreType.DMA,
        pltpu.SemaphoreType.DMA,
    ]
    if has_prev:
        scratch.insert(2, pltpu.VMEM((NCH, C), f32))

    def body(*refs):
        if has_prev:
            (feat_hbm, rowp_hbm, colp_hbm, prev_hbm, val_hbm, deg_hbm,
             rowv, colv, prevv, rbuf, cbuf, rbuf2, cbuf2, valb, degl,
             sem1, sem2) = refs
        else:
            (feat_hbm, rowp_hbm, colp_hbm, val_hbm, deg_hbm,
             rowv, colv, rbuf, cbuf, rbuf2, cbuf2, valb, degl,
             sem1, sem2) = refs
        cid = lax.axis_index("c")
        sid = lax.axis_index("s")
        wid = sid * NC + cid
        pltpu.sync_copy(rowp_hbm.at[wid], rowv)
        pltpu.sync_copy(colp_hbm.at[wid], colv)
        if has_prev:
            pltpu.sync_copy(prev_hbm.at[wid], prevv)

        zero16 = jnp.zeros((16,), f32)

        def zbody(i, carry):
            degl[pl.ds(i * 16, 16)] = zero16
            return carry
        lax.fori_loop(0, NPAD // 16, zbody, 0)

        iota = lax.iota(i32, 16)
        zv = jnp.zeros((16,), f32)
        z0 = jnp.zeros((16,), i32)

        def compute(j, rbuf_, cbuf_):
            for g in range(C // 16):
                rows16 = iota + (g * 16)

                def dot(i, accs):
                    a0, a1, a2, a3 = accs
                    kv = jnp.full((16,), i, i32) + iota
                    k0 = kv & (F - 1)
                    k1 = (kv + 1) & (F - 1)
                    k2 = (kv + 2) & (F - 1)
                    k3 = (kv + 3) & (F - 1)
                    x0 = plsc.load_gather(rbuf_, [rows16, k0])
                    y0 = plsc.load_gather(cbuf_, [rows16, k0])
                    x1 = plsc.load_gather(rbuf_, [rows16, k1])
                    y1 = plsc.load_gather(cbuf_, [rows16, k1])
                    x2 = plsc.load_gather(rbuf_, [rows16, k2])
                    y2 = plsc.load_gather(cbuf_, [rows16, k2])
                    x3 = plsc.load_gather(rbuf_, [rows16, k3])
                    y3 = plsc.load_gather(cbuf_, [rows16, k3])
                    return (a0 + x0 * y0, a1 + x1 * y1,
                            a2 + x2 * y2, a3 + x3 * y3)
                a0, a1, a2, a3 = plsc.parallel_loop(
                    0, F, 4, unroll=4, carry=(zv, zv, zv, zv))(dot)
                sims = (a0 + a1) + (a2 + a3)
                rv = rowv[j, pl.ds(g * 16, 16)]
                cv = colv[j, pl.ds(g * 16, 16)]
                m = (sims >= THR) & (rv != cv)
                if has_prev:
                    m = m & (prevv[j, pl.ds(g * 16, 16)] > 0.0)
                val = jnp.where(m, sims, 0.0)
                valb[pl.ds(g * 16, 16)] = val
                plsc.addupdate_scatter(degl, [rv], val)
            pltpu.sync_copy(valb, val_hbm.at[wid, j])

        def chunk2(jj, carry):
            j = jj * 2

            @pl.when(jj == 0)
            def _():
                pltpu.async_copy(feat_hbm.at[rowv.at[j]], rbuf, sem1)
                pltpu.async_copy(feat_hbm.at[colv.at[j]], cbuf, sem1)
            pltpu.make_async_copy(feat_hbm.at[rowv.at[j]], rbuf, sem1).wait()
            pltpu.make_async_copy(feat_hbm.at[colv.at[j]], cbuf, sem1).wait()
            pltpu.async_copy(feat_hbm.at[rowv.at[j + 1]], rbuf2, sem2)
            pltpu.async_copy(feat_hbm.at[colv.at[j + 1]], cbuf2, sem2)
            compute(j, rbuf, cbuf)
            pltpu.make_async_copy(
                feat_hbm.at[rowv.at[j + 1]], rbuf2, sem2).wait()
            pltpu.make_async_copy(
                feat_hbm.at[colv.at[j + 1]], cbuf2, sem2).wait()

            @pl.when(jj < NCH // 2 - 1)
            def _():
                pltpu.async_copy(feat_hbm.at[rowv.at[j + 2]], rbuf, sem1)
                pltpu.async_copy(feat_hbm.at[colv.at[j + 2]], cbuf, sem1)
            compute(j + 1, rbuf2, cbuf2)
            return carry
        lax.fori_loop(0, NCH // 2, chunk2, 0)
        pltpu.sync_copy(degl, deg_hbm.at[wid])

    return pl.kernel(body, out_type=out_type, mesh=_mesh,
                     compiler_params=pltpu.CompilerParams(
                         needs_layout_passes=False),
                     scratch_types=scratch)


def _make_agg_pass(D, CB):
    NCB = EPT // CB
    T = NCB // 3
    DH = D // 2
    out_type = jax.ShapeDtypeStruct((NC, NPAD, D), f32)
    scratch = [
        pltpu.VMEM((3, CB), i32),
        pltpu.VMEM((3, CB), i32),
        pltpu.VMEM((3, CB), i32),
        pltpu.VMEM((CB,), i32),
        pltpu.VMEM((CB,), i32),
        pltpu.VMEM((CB,), i32),
        pltpu.VMEM((CB, DH), i32),
        pltpu.VMEM((CB, DH), i32),
        pltpu.VMEM((CB, DH), i32),
        pltpu.VMEM((CB, D), f32),
        pltpu.VMEM((CB, D), f32),
        pltpu.VMEM((CB, D), f32),
        pltpu.VMEM((CB + 16,), f32),
        pltpu.VMEM_SHARED((NPAD, D), f32),
        pltpu.SemaphoreType.DMA,
        pltpu.SemaphoreType.DMA,
        pltpu.SemaphoreType.DMA,
    ]

    def body(feat_hbm, ivp_hbm, acc_hbm, iv0, iv1, iv2, rowb0, rowb1, rowb2,
             gi0, gi1, gi2, sbuf0, sbuf1, sbuf2, wbuf, acc_sh,
             semi, semg, sems):
        cid = lax.axis_index("c")
        sid = lax.axis_index("s")
        wid = sid * NC + cid

        zero16 = jnp.zeros((16,), f32)
        himask = jnp.full((16,), -65536, i32)

        def zrow(r, carry):
            for k in range(D // 16):
                sbuf0[r, pl.ds(k * 16, 16)] = zero16
            return carry
        lax.fori_loop(0, CB, zrow, 0)

        def zacc(t, carry):
            pltpu.sync_copy(sbuf0, acc_sh.at[pl.ds(sid * SL + t * CB, CB)])
            return carry
        lax.fori_loop(0, SL // CB, zacc, 0)
        plsc.subcore_barrier()

        def compute_scale(iv, rowb, gi, sbuf):
            for g in range(CB // 16):
                sl16 = pl.ds(g * 16, 16)
                wbuf[sl16] = plsc.bitcast(iv[2, sl16], f32)
                rowb[sl16] = iv[0, sl16]

            @plsc.parallel_loop(0, CB, 1, unroll=4)
            def scale(e):
                w16 = wbuf[pl.ds(e, 16)]
                wv = jnp.full((16,), w16[0], f32)
                for k in range(D // 32):
                    wi = gi[e, pl.ds(k * 16, 16)]
                    lo = plsc.bitcast(wi << 16, f32)
                    hi = plsc.bitcast(wi & himask, f32)
                    sbuf[e, pl.ds(k * 32, 16)] = lo * wv
                    sbuf[e, pl.ds(k * 32 + 16, 16)] = hi * wv

        def wait_iv(j, iv):
            pltpu.make_async_copy(ivp_hbm.at[wid, j], iv, semi).wait()

        def wait_gather(iv, gi):
            pltpu.make_async_copy(feat_hbm.at[iv.at[1]], gi, semg).wait()

        def wait_scatter(sbuf, rowb):
            pltpu.make_async_copy(sbuf, acc_sh.at[rowb], sems).wait()

        def triple(t, carry):
            j = 3 * t

            @pl.when(t == 0)
            def _():
                pltpu.sync_copy(ivp_hbm.at[wid, 0 * t], iv0)
                pltpu.async_copy(feat_hbm.at[iv0.at[1]], gi0, semg)
                pltpu.async_copy(ivp_hbm.at[wid, 0 * t + 1], iv1, semi)

            wait_gather(iv0, gi0)
            wait_iv(j + 1, iv1)

            @pl.when(t > 0)
            def _():
                wait_scatter(sbuf1, rowb1)
            pltpu.async_copy(feat_hbm.at[iv1.at[1]], gi1, semg)
            compute_scale(iv0, rowb0, gi0, sbuf0)
            pltpu.async_copy(sbuf0, acc_sh.at[rowb0], sems, add=True)
            pltpu.async_copy(ivp_hbm.at[wid, j + 2], iv2, semi)

            wait_gather(iv1, gi1)
            wait_iv(j + 2, iv2)

            @pl.when(t > 0)
            def _():
                wait_scatter(sbuf2, rowb2)
            pltpu.async_copy(feat_hbm.at[iv2.at[1]], gi2, semg)
            compute_scale(iv1, rowb1, gi1, sbuf1)
            pltpu.async_copy(sbuf1, acc_sh.at[rowb1], sems, add=True)

            @pl.when(t < T - 1)
            def _():
                pltpu.async_copy(ivp_hbm.at[wid, j + 3], iv0, semi)

            wait_gather(iv2, gi2)
            wait_scatter(sbuf0, rowb0)

            @pl.when(t < T - 1)
            def _():
                wait_iv(j + 3, iv0)
                pltpu.async_copy(feat_hbm.at[iv0.at[1]], gi0, semg)
                pltpu.async_copy(ivp_hbm.at[wid, j + 4], iv1, semi)
            compute_scale(iv2, rowb2, gi2, sbuf2)
            pltpu.async_copy(sbuf2, acc_sh.at[rowb2], sems, add=True)
            return carry
        lax.fori_loop(0, T, triple, 0)
        wait_scatter(sbuf1, rowb1)
        wait_scatter(sbuf2, rowb2)
        plsc.subcore_barrier()

        def dump(t, carry):
            pltpu.sync_copy(acc_sh.at[pl.ds(sid * SL + t * CB, CB)], sbuf0)
            pltpu.sync_copy(sbuf0, acc_hbm.at[cid, pl.ds(sid * SL + t * CB, CB)])
            return carry
        lax.fori_loop(0, SL // CB, dump, 0)

    return pl.kernel(body, out_type=out_type, mesh=_mesh,
                     compiler_params=pltpu.CompilerParams(
                         needs_layout_passes=False,
                         use_tc_tiling_on_sc=False),
                     scratch_types=scratch)


_sim_pass1 = _make_sim_pass(False)
_sim_pass2 = _make_sim_pass(True)
CB1 = 64
CB2 = 128
_agg_pass1 = _make_agg_pass(F, CB1)
_agg_pass2 = _make_agg_pass(NCLASS, CB2)


def _tc1_body(x_ref, xn_ref):
    x = x_ref[...]
    nr = jnp.sqrt(jnp.sum(x * x, axis=1, keepdims=True))
    xn_ref[...] = x / jnp.maximum(nr, 1e-8)


def _tc1(x):
    return pl.pallas_call(
        _tc1_body,
        out_shape=jax.ShapeDtypeStruct((N, F), f32),
    )(x)


def _tc2_body(degp_ref, x_ref, dinv_ref, z1_ref):
    deg = jnp.sum(degp_ref[...], axis=0)[:, None] + 1.0
    dinv = lax.rsqrt(deg)
    dinv_ref[...] = dinv
    z1_ref[...] = (dinv[:N] * x_ref[...]).astype(jnp.bfloat16)


def _tc2(degp, x):
    return pl.pallas_call(
        _tc2_body,
        out_shape=(jax.ShapeDtypeStruct((NPAD, 1), f32),
                   jax.ShapeDtypeStruct((N, F), jnp.bfloat16)),
    )(degp, x)


def _tc3_body(accp_ref, x_ref, dinv_ref, W1_ref, b1_ref, hn_ref, nr2c_ref):
    dinv = dinv_ref[...][:N]
    acc = accp_ref[0, :N] + accp_ref[1, :N]
    pre = dinv * acc + (dinv * dinv) * x_ref[...]
    h = jnp.maximum(jnp.dot(pre, W1_ref[...],
                            preferred_element_type=f32) + b1_ref[...], 0.0)
    nr2 = jnp.sqrt(jnp.sum(h * h, axis=1, keepdims=True))
    nr2c = jnp.maximum(nr2, 1e-8)
    hn_ref[...] = h / nr2c
    nr2c_ref[...] = jnp.concatenate(
        [nr2c, jnp.ones((NPAD - N, 1), f32)], axis=0)


def _tc3(accp, x, dinv1, W1, b1):
    return pl.pallas_call(
        _tc3_body,
        out_shape=(jax.ShapeDtypeStruct((N, F), f32),
                   jax.ShapeDtypeStruct((NPAD, 1), f32)),
    )(accp, x, dinv1, W1, b1)


def _tc4_body(degp_ref, nr2c_ref, hn_ref, W2_ref, dinv_ref, z2_ref, z2b_ref):
    deg = jnp.sum(degp_ref[...], axis=0)[:, None] + 1.0
    dinv = lax.rsqrt(deg)
    dinv_ref[...] = dinv
    scale = (dinv * nr2c_ref[...])[:N]
    z2 = jnp.dot(scale * hn_ref[...], W2_ref[...],
                 preferred_element_type=f32)
    z2_ref[...] = z2
    z2b_ref[...] = z2.astype(jnp.bfloat16)


def _tc4(degp, nr2c, hn, W2):
    return pl.pallas_call(
        _tc4_body,
        out_shape=(jax.ShapeDtypeStruct((NPAD, 1), f32),
                   jax.ShapeDtypeStruct((N, NCLASS), f32),
                   jax.ShapeDtypeStruct((N, NCLASS), jnp.bfloat16)),
    )(degp, nr2c, hn, W2)


def _tc5_body(accp_ref, z2_ref, dinv_ref, b2_ref, out_ref):
    acc = accp_ref[0, :N] + accp_ref[1, :N] + z2_ref[...]
    out_ref[...] = dinv_ref[...][:N] * acc + b2_ref[...]


def _tc5(accp, z2, dinv2, b2):
    return pl.pallas_call(
        _tc5_body,
        out_shape=jax.ShapeDtypeStruct((N, NCLASS), f32),
    )(accp, z2, dinv2, b2)


def _make_ivp(rowf, colf, val, CB):
    NCB = EPT // CB
    r = rowf.reshape(NW, NCB, 1, CB)
    c = colf.reshape(NW, NCB, 1, CB)
    v = lax.bitcast_convert_type(val, i32).reshape(NW, NCB, 1, CB)
    return jnp.concatenate([r, c, v], axis=2)


def _pack_cols(zb):
    D = zb.shape[1]
    u = lax.bitcast_convert_type(zb, jnp.uint16).reshape(N, D // 32, 2, 16)
    lo = u[:, :, 0, :].astype(jnp.uint32)
    hi = u[:, :, 1, :].astype(jnp.uint32)
    w = lo | (hi << 16)
    return lax.bitcast_convert_type(w, i32).reshape(N, D // 2)


def kernel(x, adj, W1, b1, W2, b2):
    E = adj.shape[1]
    pad = EPAD - E
    row = adj[0]
    col = adj[1]
    zpad = jnp.arange(pad, dtype=i32) % N
    rowf = jnp.concatenate([row, zpad])
    colf = jnp.concatenate([col, zpad])
    rowp = rowf.reshape(NW, NCH, C)
    colp = colf.reshape(NW, NCH, C)

    xn = _tc1(x)
    val1, deg1p = _sim_pass1(xn, rowp, colp)
    dinv1, z1b = _tc2(deg1p, x)
    acc1p = _agg_pass1(_pack_cols(z1b), _make_ivp(rowf, colf, val1, CB1))
    hn, nr2c = _tc3(acc1p, x, dinv1, W1, b1)
    val2, deg2p = _sim_pass2(hn, rowp, colp, val1)
    dinv2, z2, z2b = _tc4(deg2p, nr2c, hn, W2)
    acc2p = _agg_pass2(_pack_cols(z2b), _make_ivp(rowf, colf, val2, CB2))
    return _tc5(acc2p, z2, dinv2, b2)


# speedup vs baseline: 23.7220x; 1.0230x over previous
import functools

import jax
import jax.numpy as jnp
from jax import lax
from jax.experimental import pallas as pl
from jax.experimental.pallas import tpu as pltpu
from jax.experimental.pallas import tpu_sc as plsc

N = 10000
NPAD = 10240
F = 128
NCLASS = 64
THR = 0.1
NC = 2
NS = 16
NW = NC * NS
C = 128
EPT = 10752
NCH = EPT // C
EPAD = NW * EPT
SL = NPAD // NS
f32 = jnp.float32
i32 = jnp.int32

_mesh = plsc.VectorSubcoreMesh(core_axis_name="c", subcore_axis_name="s")


def _make_sim_pass(has_prev):
    out_type = (jax.ShapeDtypeStruct((NW, NCH, C), f32),
                jax.ShapeDtypeStruct((NW, NPAD), f32))
    scratch = [
        pltpu.VMEM((NCH, C), i32),
        pltpu.VMEM((NCH, C), i32),
        pltpu.VMEM((C, F), f32),
        pltpu.VMEM((C, F), f32),
        pltpu.VMEM((C, F), f32),
        pltpu.VMEM((C, F), f32),
        pltpu.VMEM((C,), f32),
        pltpu.VMEM((NPAD,), f32),
        pltpu.SemaphoreType.DMA,
        pltpu.SemaphoreType.DMA,
    ]
    if has_prev:
        scratch.insert(2, pltpu.VMEM((NCH, C), f32))

    def body(*refs):
        if has_prev:
            (feat_hbm, rowp_hbm, colp_hbm, prev_hbm, val_hbm, deg_hbm,
             rowv, colv, prevv, rbuf, cbuf, rbuf2, cbuf2, valb, degl,
             sem1, sem2) = refs
        else:
            (feat_hbm, rowp_hbm, colp_hbm, val_hbm, deg_hbm,
             rowv, colv, rbuf, cbuf, rbuf2, cbuf2, valb, degl,
             sem1, sem2) = refs
        cid = lax.axis_index("c")
        sid = lax.axis_index("s")
        wid = sid * NC + cid
        pltpu.sync_copy(rowp_hbm.at[wid], rowv)
        pltpu.sync_copy(colp_hbm.at[wid], colv)
        if has_prev:
            pltpu.sync_copy(prev_hbm.at[wid], prevv)

        zero16 = jnp.zeros((16,), f32)

        def zbody(i, carry):
            degl[pl.ds(i * 16, 16)] = zero16
            return carry
        lax.fori_loop(0, NPAD // 16, zbody, 0)

        iota = lax.iota(i32, 16)
        zv = jnp.zeros((16,), f32)
        z0 = jnp.zeros((16,), i32)

        def compute(j, rbuf_, cbuf_):
            for g in range(C // 16):
                rows16 = iota + (g * 16)

                def dot(i, accs):
                    a0, a1, a2, a3 = accs
                    kv = jnp.full((16,), i, i32) + iota
                    k0 = kv & (F - 1)
                    k1 = (kv + 1) & (F - 1)
                    k2 = (kv + 2) & (F - 1)
                    k3 = (kv + 3) & (F - 1)
                    x0 = plsc.load_gather(rbuf_, [rows16, k0])
                    y0 = plsc.load_gather(cbuf_, [rows16, k0])
                    x1 = plsc.load_gather(rbuf_, [rows16, k1])
                    y1 = plsc.load_gather(cbuf_, [rows16, k1])
                    x2 = plsc.load_gather(rbuf_, [rows16, k2])
                    y2 = plsc.load_gather(cbuf_, [rows16, k2])
                    x3 = plsc.load_gather(rbuf_, [rows16, k3])
                    y3 = plsc.load_gather(cbuf_, [rows16, k3])
                    return (a0 + x0 * y0, a1 + x1 * y1,
                            a2 + x2 * y2, a3 + x3 * y3)
                a0, a1, a2, a3 = plsc.parallel_loop(
                    0, F, 4, unroll=4, carry=(zv, zv, zv, zv))(dot)
                sims = (a0 + a1) + (a2 + a3)
                rv = rowv[j, pl.ds(g * 16, 16)]
                cv = colv[j, pl.ds(g * 16, 16)]
                m = (sims >= THR) & (rv != cv)
                if has_prev:
                    m = m & (prevv[j, pl.ds(g * 16, 16)] > 0.0)
                val = jnp.where(m, sims, 0.0)
                valb[pl.ds(g * 16, 16)] = val
                plsc.addupdate_scatter(degl, [rv], val)
            pltpu.sync_copy(valb, val_hbm.at[wid, j])

        def chunk2(jj, carry):
            j = jj * 2

            @pl.when(jj == 0)
            def _():
                pltpu.async_copy(feat_hbm.at[rowv.at[j]], rbuf, sem1)
                pltpu.async_copy(feat_hbm.at[colv.at[j]], cbuf, sem1)
            pltpu.make_async_copy(feat_hbm.at[rowv.at[j]], rbuf, sem1).wait()
            pltpu.make_async_copy(feat_hbm.at[colv.at[j]], cbuf, sem1).wait()
            pltpu.async_copy(feat_hbm.at[rowv.at[j + 1]], rbuf2, sem2)
            pltpu.async_copy(feat_hbm.at[colv.at[j + 1]], cbuf2, sem2)
            compute(j, rbuf, cbuf)
            pltpu.make_async_copy(
                feat_hbm.at[rowv.at[j + 1]], rbuf2, sem2).wait()
            pltpu.make_async_copy(
                feat_hbm.at[colv.at[j + 1]], cbuf2, sem2).wait()

            @pl.when(jj < NCH // 2 - 1)
            def _():
                pltpu.async_copy(feat_hbm.at[rowv.at[j + 2]], rbuf, sem1)
                pltpu.async_copy(feat_hbm.at[colv.at[j + 2]], cbuf, sem1)
            compute(j + 1, rbuf2, cbuf2)
            return carry
        lax.fori_loop(0, NCH // 2, chunk2, 0)
        pltpu.sync_copy(degl, deg_hbm.at[wid])

    return pl.kernel(body, out_type=out_type, mesh=_mesh,
                     compiler_params=pltpu.CompilerParams(
                         needs_layout_passes=False),
                     scratch_types=scratch)


def _make_agg_pass(D, CB):
    NCB = EPT // CB
    T = NCB // 3
    DH = D // 2
    out_type = jax.ShapeDtypeStruct((NC, NPAD, D), f32)
    scratch = [
        pltpu.VMEM((3, CB), i32),
        pltpu.VMEM((3, CB), i32),
        pltpu.VMEM((3, CB), i32),
        pltpu.VMEM((CB,), i32),
        pltpu.VMEM((CB,), i32),
        pltpu.VMEM((CB,), i32),
        pltpu.VMEM((CB, DH), i32),
        pltpu.VMEM((CB, DH), i32),
        pltpu.VMEM((CB, DH), i32),
        pltpu.VMEM((CB, D), f32),
        pltpu.VMEM((CB, D), f32),
        pltpu.VMEM((CB, D), f32),
        pltpu.VMEM((CB + 16,), f32),
        pltpu.VMEM_SHARED((NPAD, D), f32),
        pltpu.SemaphoreType.DMA,
        pltpu.SemaphoreType.DMA,
        pltpu.SemaphoreType.DMA,
    ]

    def body(feat_hbm, ivp_hbm, acc_hbm, iv0, iv1, iv2, rowb0, rowb1, rowb2,
             gi0, gi1, gi2, sbuf0, sbuf1, sbuf2, wbuf, acc_sh,
             semi, semg, sems):
        cid = lax.axis_index("c")
        sid = lax.axis_index("s")
        wid = sid * NC + cid

        zero16 = jnp.zeros((16,), f32)
        himask = jnp.full((16,), -65536, i32)

        def zrow(r, carry):
            for k in range(D // 16):
                sbuf0[r, pl.ds(k * 16, 16)] = zero16
            return carry
        lax.fori_loop(0, CB, zrow, 0)

        def zacc(t, carry):
            pltpu.sync_copy(sbuf0, acc_sh.at[pl.ds(sid * SL + t * CB, CB)])
            return carry
        lax.fori_loop(0, SL // CB, zacc, 0)
        plsc.subcore_barrier()

        def compute_scale(iv, rowb, gi, sbuf):
            for g in range(CB // 16):
                sl16 = pl.ds(g * 16, 16)
                wbuf[sl16] = plsc.bitcast(iv[2, sl16], f32)
                rowb[sl16] = iv[0, sl16]

            @plsc.parallel_loop(0, CB, 1, unroll=4)
            def scale(e):
                w16 = wbuf[pl.ds(e, 16)]
                wv = jnp.full((16,), w16[0], f32)
                for k in range(D // 32):
                    wi = gi[e, pl.ds(k * 16, 16)]
                    lo = plsc.bitcast(wi << 16, f32)
                    hi = plsc.bitcast(wi & himask, f32)
                    sbuf[e, pl.ds(k * 32, 16)] = lo * wv
                    sbuf[e, pl.ds(k * 32 + 16, 16)] = hi * wv

        def wait_iv(j, iv):
            pltpu.make_async_copy(ivp_hbm.at[wid, j], iv, semi).wait()

        def wait_gather(iv, gi):
            pltpu.make_async_copy(feat_hbm.at[iv.at[1]], gi, semg).wait()

        def wait_scatter(sbuf, rowb):
            pltpu.make_async_copy(sbuf, acc_sh.at[rowb], sems).wait()

        def triple(t, carry):
            j = 3 * t

            @pl.when(t == 0)
            def _():
                pltpu.sync_copy(ivp_hbm.at[wid, 0 * t], iv0)
                pltpu.async_copy(feat_hbm.at[iv0.at[1]], gi0, semg)
                pltpu.async_copy(ivp_hbm.at[wid, 0 * t + 1], iv1, semi)

            wait_gather(iv0, gi0)
            wait_iv(j + 1, iv1)

            @pl.when(t > 0)
            def _():
                wait_scatter(sbuf1, rowb1)
            pltpu.async_copy(feat_hbm.at[iv1.at[1]], gi1, semg)
            compute_scale(iv0, rowb0, gi0, sbuf0)
            pltpu.async_copy(sbuf0, acc_sh.at[rowb0], sems, add=True)
            pltpu.async_copy(ivp_hbm.at[wid, j + 2], iv2, semi)

            wait_gather(iv1, gi1)
            wait_iv(j + 2, iv2)

            @pl.when(t > 0)
            def _():
                wait_scatter(sbuf2, rowb2)
            pltpu.async_copy(feat_hbm.at[iv2.at[1]], gi2, semg)
            compute_scale(iv1, rowb1, gi1, sbuf1)
            pltpu.async_copy(sbuf1, acc_sh.at[rowb1], sems, add=True)

            @pl.when(t < T - 1)
            def _():
                pltpu.async_copy(ivp_hbm.at[wid, j + 3], iv0, semi)

            wait_gather(iv2, gi2)
            wait_scatter(sbuf0, rowb0)

            @pl.when(t < T - 1)
            def _():
                wait_iv(j + 3, iv0)
                pltpu.async_copy(feat_hbm.at[iv0.at[1]], gi0, semg)
                pltpu.async_copy(ivp_hbm.at[wid, j + 4], iv1, semi)
            compute_scale(iv2, rowb2, gi2, sbuf2)
            pltpu.async_copy(sbuf2, acc_sh.at[rowb2], sems, add=True)
            return carry
        lax.fori_loop(0, T, triple, 0)
        wait_scatter(sbuf1, rowb1)
        wait_scatter(sbuf2, rowb2)
        plsc.subcore_barrier()

        def dump(t, carry):
            pltpu.sync_copy(acc_sh.at[pl.ds(sid * SL + t * CB, CB)], sbuf0)
            pltpu.sync_copy(sbuf0, acc_hbm.at[cid, pl.ds(sid * SL + t * CB, CB)])
            return carry
        lax.fori_loop(0, SL // CB, dump, 0)

    return pl.kernel(body, out_type=out_type, mesh=_mesh,
                     compiler_params=pltpu.CompilerParams(
                         needs_layout_passes=False,
                         use_tc_tiling_on_sc=False),
                     scratch_types=scratch)


_sim_pass1 = _make_sim_pass(False)
_sim_pass2 = _make_sim_pass(True)
CB1 = 64
CB2 = 128
_agg_pass1 = _make_agg_pass(F, CB1)
_agg_pass2 = _make_agg_pass(NCLASS, CB2)


def _tc1_body(x_ref, xn_ref):
    x = x_ref[...]
    nr = jnp.sqrt(jnp.sum(x * x, axis=1, keepdims=True))
    xn_ref[...] = x / jnp.maximum(nr, 1e-8)


def _tc1(x):
    return pl.pallas_call(
        _tc1_body,
        out_shape=jax.ShapeDtypeStruct((N, F), f32),
    )(x)


def _pack_tc(z):
    u = lax.bitcast_convert_type(z.astype(jnp.bfloat16), jnp.uint16)
    words = []
    for k in range(z.shape[1] // 32):
        lo = u[:, k * 32:k * 32 + 16].astype(jnp.uint32)
        hi = u[:, k * 32 + 16:k * 32 + 32].astype(jnp.uint32)
        words.append(lo | (hi << 16))
    return lax.bitcast_convert_type(jnp.concatenate(words, axis=1), i32)


def _tc2_body(degp_ref, x_ref, dinv_ref, z1p_ref):
    deg = jnp.sum(degp_ref[...], axis=0)[:, None] + 1.0
    dinv = lax.rsqrt(deg)
    dinv_ref[...] = dinv
    z1p_ref[...] = _pack_tc(dinv[:N] * x_ref[...])


def _tc2(degp, x):
    return pl.pallas_call(
        _tc2_body,
        out_shape=(jax.ShapeDtypeStruct((NPAD, 1), f32),
                   jax.ShapeDtypeStruct((N, F // 2), i32)),
    )(degp, x)


def _tc3_body(accp_ref, x_ref, dinv_ref, W1_ref, b1_ref, hn_ref, nr2c_ref):
    dinv = dinv_ref[...][:N]
    acc = accp_ref[0, :N] + accp_ref[1, :N]
    pre = dinv * acc + (dinv * dinv) * x_ref[...]
    h = jnp.maximum(jnp.dot(pre, W1_ref[...],
                            preferred_element_type=f32) + b1_ref[...], 0.0)
    nr2 = jnp.sqrt(jnp.sum(h * h, axis=1, keepdims=True))
    nr2c = jnp.maximum(nr2, 1e-8)
    hn_ref[...] = h / nr2c
    nr2c_ref[...] = jnp.concatenate(
        [nr2c, jnp.ones((NPAD - N, 1), f32)], axis=0)


def _tc3(accp, x, dinv1, W1, b1):
    return pl.pallas_call(
        _tc3_body,
        out_shape=(jax.ShapeDtypeStruct((N, F), f32),
                   jax.ShapeDtypeStruct((NPAD, 1), f32)),
    )(accp, x, dinv1, W1, b1)


def _tc4_body(degp_ref, nr2c_ref, hn_ref, W2_ref, dinv_ref, z2_ref, z2p_ref):
    deg = jnp.sum(degp_ref[...], axis=0)[:, None] + 1.0
    dinv = lax.rsqrt(deg)
    dinv_ref[...] = dinv
    scale = (dinv * nr2c_ref[...])[:N]
    z2 = jnp.dot(scale * hn_ref[...], W2_ref[...],
                 preferred_element_type=f32)
    z2_ref[...] = z2
    z2p_ref[...] = _pack_tc(z2)


def _tc4(degp, nr2c, hn, W2):
    return pl.pallas_call(
        _tc4_body,
        out_shape=(jax.ShapeDtypeStruct((NPAD, 1), f32),
                   jax.ShapeDtypeStruct((N, NCLASS), f32),
                   jax.ShapeDtypeStruct((N, NCLASS // 2), i32)),
    )(degp, nr2c, hn, W2)


def _tc5_body(accp_ref, z2_ref, dinv_ref, b2_ref, out_ref):
    acc = accp_ref[0, :N] + accp_ref[1, :N] + z2_ref[...]
    out_ref[...] = dinv_ref[...][:N] * acc + b2_ref[...]


def _tc5(accp, z2, dinv2, b2):
    return pl.pallas_call(
        _tc5_body,
        out_shape=jax.ShapeDtypeStruct((N, NCLASS), f32),
    )(accp, z2, dinv2, b2)


def _make_ivp(rowf, colf, val, CB):
    NCB = EPT // CB
    r = rowf.reshape(NW, NCB, 1, CB)
    c = colf.reshape(NW, NCB, 1, CB)
    v = lax.bitcast_convert_type(val, i32).reshape(NW, NCB, 1, CB)
    return jnp.concatenate([r, c, v], axis=2)


def kernel(x, adj, W1, b1, W2, b2):
    E = adj.shape[1]
    pad = EPAD - E
    row = adj[0]
    col = adj[1]
    zpad = jnp.arange(pad, dtype=i32) % N
    rowf = jnp.concatenate([row, zpad])
    colf = jnp.concatenate([col, zpad])
    rowp = rowf.reshape(NW, NCH, C)
    colp = colf.reshape(NW, NCH, C)

    xn = _tc1(x)
    val1, deg1p = _sim_pass1(xn, rowp, colp)
    dinv1, z1p = _tc2(deg1p, x)
    acc1p = _agg_pass1(z1p, _make_ivp(rowf, colf, val1, CB1))
    hn, nr2c = _tc3(acc1p, x, dinv1, W1, b1)
    val2, deg2p = _sim_pass2(hn, rowp, colp, val1)
    dinv2, z2, z2p = _tc4(deg2p, nr2c, hn, W2)
    acc2p = _agg_pass2(z2p, _make_ivp(rowf, colf, val2, CB2))
    return _tc5(acc2p, z2, dinv2, b2)
